# Initial kernel scaffold; baseline (speedup 1.0000x reference)
import jax
import jax.numpy as jnp
from jax.experimental import pallas as pl


def kernel(x, edge_index, W1, b1, We1, be1, We2, be2, Wc1, bc1, Wc2, bc2):
    raise NotImplementedError("write your pallas kernel here")


# speedup vs baseline: 5.6628x; 5.6628x over previous
import functools

import jax
import jax.numpy as jnp
from jax import lax
from jax.experimental import pallas as pl
from jax.experimental.pallas import tpu as pltpu
from jax.experimental.pallas import tpu_sc as plsc

N = 10000
E = 320000
D_IN = 128
DH = 64
DO = 32

NC = 2
NS = 16
NW = NC * NS
EPW = E // NW
CH = 128
NFULL = EPW // CH
TAIL = EPW - NFULL * CH
NP2 = 10240
RSUB = NP2 // NS
ZC = 128
NZC = RSUB // ZC

RNG = 320
NPAD = NW * RNG
SCH = 2000
NSCH = E // SCH
CAP = 16384
NEG = -3.0e38


def _mesh():
  return plsc.VectorSubcoreMesh(core_axis_name="c", subcore_axis_name="s")


_SC_PARAMS = pltpu.CompilerParams(use_tc_tiling_on_sc=False, needs_layout_passes=False)


def _sc_hist(dst):
  @functools.partial(
      pl.kernel,
      mesh=_mesh(),
      compiler_params=_SC_PARAMS,
      out_type=jax.ShapeDtypeStruct((NC, NP2, 16), jnp.float32),
      scratch_types=[
          pltpu.VMEM((CH,), jnp.int32),
          pltpu.VMEM((TAIL,), jnp.int32),
          pltpu.VMEM((CH, 16), jnp.float32),
          pltpu.VMEM((ZC, 16), jnp.float32),
          pltpu.VMEM_SHARED((NP2, 16), jnp.float32),
      ],
  )
  def k(dst_hbm, out_hbm, idxb, idxt, ones, zbuf, hist):
    c = lax.axis_index("c")
    s = lax.axis_index("s")
    wid = s * NC + c
    one = jnp.ones((16,), jnp.float32)
    zero = jnp.zeros((16,), jnp.float32)

    def fill_ones(i, _):
      ones[i, :] = one
      return _

    lax.fori_loop(0, CH, fill_ones, 0)

    def fill_z(i, _):
      zbuf[i, :] = zero
      return _

    lax.fori_loop(0, ZC, fill_z, 0)

    def zc(i, _):
      pltpu.sync_copy(zbuf, hist.at[pl.ds(s * RSUB + i * ZC, ZC)])
      return _

    lax.fori_loop(0, NZC, zc, 0)
    plsc.subcore_barrier()

    base = wid * EPW

    def step(j, _):
      pltpu.sync_copy(dst_hbm.at[pl.ds(base + j * CH, CH)], idxb)
      pltpu.sync_copy(ones, hist.at[idxb], add=True)
      return _

    lax.fori_loop(0, NFULL, step, 0)
    pltpu.sync_copy(dst_hbm.at[pl.ds(base + NFULL * CH, TAIL)], idxt)
    pltpu.sync_copy(ones.at[pl.ds(0, TAIL)], hist.at[idxt], add=True)
    plsc.subcore_barrier()
    pltpu.sync_copy(hist.at[pl.ds(s * RSUB, RSUB)],
                    out_hbm.at[c, pl.ds(s * RSUB, RSUB)])

  return k(dst)


def _tc_b(x, W1, degp):
  RB = 1000
  nb = N // RB

  def body(x_ref, w_ref, d_ref, h_ref, g_ref, dv_ref):
    xx = x_ref[...]
    h = jnp.dot(xx, w_ref[...], preferred_element_type=jnp.float32)
    d = d_ref[...]
    deg = d[0, :, 0] + d[1, :, 0] + 1.0
    dinv = lax.rsqrt(deg)
    h_ref[...] = h
    g_ref[...] = h * dinv[:, None]
    dv_ref[...] = dinv[:, None]

  return pl.pallas_call(
      body,
      grid=(nb,),
      in_specs=[
          pl.BlockSpec((RB, D_IN), lambda i: (i, 0)),
          pl.BlockSpec((D_IN, DH), lambda i: (0, 0)),
          pl.BlockSpec((NC, RB, 16), lambda i: (0, i, 0)),
      ],
      out_specs=[
          pl.BlockSpec((RB, DH), lambda i: (i, 0)),
          pl.BlockSpec((RB, DH), lambda i: (i, 0)),
          pl.BlockSpec((RB, 1), lambda i: (i, 0)),
      ],
      out_shape=[
          jax.ShapeDtypeStruct((N, DH), jnp.float32),
          jax.ShapeDtypeStruct((N, DH), jnp.float32),
          jax.ShapeDtypeStruct((N, 1), jnp.float32),
      ],
  )(x, W1, degp)


def _sc_scatter_add(src, dst, g):
  @functools.partial(
      pl.kernel,
      mesh=_mesh(),
      compiler_params=_SC_PARAMS,
      out_type=jax.ShapeDtypeStruct((NC, NP2, DH), jnp.float32),
      scratch_types=[
          pltpu.VMEM((CH,), jnp.int32),
          pltpu.VMEM((CH,), jnp.int32),
          pltpu.VMEM((CH, DH), jnp.float32),
          pltpu.VMEM((TAIL,), jnp.int32),
          pltpu.VMEM((TAIL,), jnp.int32),
          pltpu.VMEM((TAIL, DH), jnp.float32),
          pltpu.VMEM((ZC, DH), jnp.float32),
          pltpu.SemaphoreType.DMA,
          pltpu.VMEM_SHARED((NP2, DH), jnp.float32),
      ],
  )
  def k(src_hbm, dst_hbm, g_hbm, out_hbm, sbuf, dbuf, gbuf, sbt, dbt, gbt,
        zb, sem, acc):
    c = lax.axis_index("c")
    s = lax.axis_index("s")
    wid = s * NC + c
    zero = jnp.zeros((16,), jnp.float32)

    def zrow(i, _):
      for kk in range(DH // 16):
        zb[i, pl.ds(kk * 16, 16)] = zero
      return _

    lax.fori_loop(0, ZC, zrow, 0)

    def zc(i, _):
      pltpu.sync_copy(zb, acc.at[pl.ds(s * RSUB + i * ZC, ZC)])
      return _

    lax.fori_loop(0, NZC, zc, 0)
    plsc.subcore_barrier()

    base = wid * EPW

    def step(j, _):
      pltpu.sync_copy(src_hbm.at[pl.ds(base + j * CH, CH)], sbuf)
      pltpu.sync_copy(dst_hbm.at[pl.ds(base + j * CH, CH)], dbuf)
      pltpu.async_copy(g_hbm.at[sbuf], gbuf, sem).wait()
      pltpu.sync_copy(gbuf, acc.at[dbuf], add=True)
      return _

    lax.fori_loop(0, NFULL, step, 0)
    pltpu.sync_copy(src_hbm.at[pl.ds(base + NFULL * CH, TAIL)], sbt)
    pltpu.sync_copy(dst_hbm.at[pl.ds(base + NFULL * CH, TAIL)], dbt)
    pltpu.async_copy(g_hbm.at[sbt], gbt, sem).wait()
    pltpu.sync_copy(gbt, acc.at[dbt], add=True)
    plsc.subcore_barrier()

    def dump(i, _):
      pltpu.sync_copy(acc.at[pl.ds(s * RSUB + i * ZC, ZC)],
                      out_hbm.at[c, pl.ds(s * RSUB + i * ZC, ZC)])
      return _

    lax.fori_loop(0, NZC, dump, 0)

  return k(src, dst, g)


def _tc_d(accp, h, dinv, b1, We1, be1):
  RB = 1000
  nb = N // RB

  def body(a_ref, h_ref, dv_ref, b1_ref, w_ref, be_ref, p_ref, q_ref):
    acc = a_ref[0] + a_ref[1]
    dv = dv_ref[...]
    hh = h_ref[...]
    h1 = jnp.maximum(dv * acc + dv * dv * hh + b1_ref[...], 0.0)
    A = w_ref[:DH, :]
    B = w_ref[DH:, :]
    p_ref[...] = jnp.dot(h1, A - B, preferred_element_type=jnp.float32) + be_ref[...]
    q_ref[...] = jnp.dot(h1, B, preferred_element_type=jnp.float32)

  return pl.pallas_call(
      body,
      grid=(nb,),
      in_specs=[
          pl.BlockSpec((NC, RB, DH), lambda i: (0, i, 0)),
          pl.BlockSpec((RB, DH), lambda i: (i, 0)),
          pl.BlockSpec((RB, 1), lambda i: (i, 0)),
          pl.BlockSpec((1, DH), lambda i: (0, 0)),
          pl.BlockSpec((2 * DH, DH), lambda i: (0, 0)),
          pl.BlockSpec((1, DH), lambda i: (0, 0)),
      ],
      out_specs=[
          pl.BlockSpec((RB, DH), lambda i: (i, 0)),
          pl.BlockSpec((RB, DH), lambda i: (i, 0)),
      ],
      out_shape=[
          jax.ShapeDtypeStruct((N, DH), jnp.float32),
          jax.ShapeDtypeStruct((N, DH), jnp.float32),
      ],
  )(accp, h, dinv, b1.reshape(1, DH), We1, be1.reshape(1, DH))


def _sc_edge_u(src, dst, P, Q):
  @functools.partial(
      pl.kernel,
      mesh=_mesh(),
      compiler_params=_SC_PARAMS,
      out_type=jax.ShapeDtypeStruct((E, DH), jnp.float32),
      scratch_types=[
          pltpu.VMEM((CH,), jnp.int32),
          pltpu.VMEM((CH,), jnp.int32),
          pltpu.VMEM((CH, DH), jnp.float32),
          pltpu.VMEM((CH, DH), jnp.float32),
          pltpu.VMEM((TAIL,), jnp.int32),
          pltpu.VMEM((TAIL,), jnp.int32),
          pltpu.VMEM((TAIL, DH), jnp.float32),
          pltpu.VMEM((TAIL, DH), jnp.float32),
          pltpu.SemaphoreType.DMA,
          pltpu.SemaphoreType.DMA,
      ],
  )
  def k(src_hbm, dst_hbm, p_hbm, q_hbm, u_hbm, sbuf, dbuf, pbuf, qbuf,
        sbt, dbt, pbt, qbt, sem1, sem2):
    c = lax.axis_index("c")
    s = lax.axis_index("s")
    wid = s * NC + c
    base = wid * EPW

    def step(j, _):
      pltpu.sync_copy(dst_hbm.at[pl.ds(base + j * CH, CH)], dbuf)
      pltpu.sync_copy(src_hbm.at[pl.ds(base + j * CH, CH)], sbuf)
      cp1 = pltpu.async_copy(p_hbm.at[dbuf], pbuf, sem1)
      cp2 = pltpu.async_copy(q_hbm.at[sbuf], qbuf, sem2)
      cp1.wait()
      cp2.wait()

      def addrow(r, _):
        for kk in range(DH // 16):
          sl = pl.ds(kk * 16, 16)
          pbuf[r, sl] = pbuf[r, sl] + qbuf[r, sl]
        return _

      lax.fori_loop(0, CH, addrow, 0)
      pltpu.sync_copy(pbuf, u_hbm.at[pl.ds(base + j * CH, CH)])
      return _

    lax.fori_loop(0, NFULL, step, 0)

    pltpu.sync_copy(dst_hbm.at[pl.ds(base + NFULL * CH, TAIL)], dbt)
    pltpu.sync_copy(src_hbm.at[pl.ds(base + NFULL * CH, TAIL)], sbt)
    cp1 = pltpu.async_copy(p_hbm.at[dbt], pbt, sem1)
    cp2 = pltpu.async_copy(q_hbm.at[sbt], qbt, sem2)
    cp1.wait()
    cp2.wait()

    def addrow_t(r, _):
      for kk in range(DH // 16):
        sl = pl.ds(kk * 16, 16)
        pbt[r, sl] = pbt[r, sl] + qbt[r, sl]
      return _

    lax.fori_loop(0, TAIL, addrow_t, 0)
    pltpu.sync_copy(pbt, u_hbm.at[pl.ds(base + NFULL * CH, TAIL)])

  return k(src, dst, P, Q)


def _tc_f(U, We2, be2):
  RB = 2000
  nb = E // RB

  def body(u_ref, w_ref, b_ref, m_ref):
    u = jnp.maximum(u_ref[...], 0.0)
    m_ref[...] = jnp.dot(u, w_ref[...], preferred_element_type=jnp.float32) + b_ref[...]

  return pl.pallas_call(
      body,
      grid=(nb,),
      in_specs=[
          pl.BlockSpec((RB, DH), lambda i: (i, 0)),
          pl.BlockSpec((DH, DO), lambda i: (0, 0)),
          pl.BlockSpec((1, DO), lambda i: (0, 0)),
      ],
      out_specs=pl.BlockSpec((RB, DO), lambda i: (i, 0)),
      out_shape=jax.ShapeDtypeStruct((E, DO), jnp.float32),
  )(U, We2, be2.reshape(1, DO))


def _sc_segmax(dst, M):
  @functools.partial(
      pl.kernel,
      mesh=_mesh(),
      compiler_params=_SC_PARAMS,
      out_type=jax.ShapeDtypeStruct((NPAD, DO), jnp.float32),
      scratch_types=[
          pltpu.VMEM((SCH,), jnp.int32),
          pltpu.VMEM((CAP,), jnp.int32),
          pltpu.VMEM((CAP,), jnp.int32),
          pltpu.VMEM((RNG, DO), jnp.float32),
          pltpu.VMEM((CH, DO), jnp.float32),
          pltpu.SemaphoreType.DMA,
      ],
  )
  def k(dst_hbm, m_hbm, ef_hbm, dsb, L, Ld, acc, mbuf, sem):
    c = lax.axis_index("c")
    s = lax.axis_index("s")
    wid = s * NC + c
    lo = wid * RNG
    hi = lo + RNG
    izero = jnp.zeros((16,), jnp.int32)
    negv = jnp.full((16,), NEG, jnp.float32)

    def initL(i, _):
      L[pl.ds(i * 16, 16)] = izero
      return _

    lax.fori_loop(0, CAP // 16, initL, 0)

    def inita(i, _):
      for kk in range(DO // 16):
        acc[i, pl.ds(kk * 16, 16)] = negv
      return _

    lax.fori_loop(0, RNG, inita, 0)

    def outer(jc, cnt):
      pltpu.sync_copy(dst_hbm.at[pl.ds(jc * SCH, SCH)], dsb)

      def inner(i, cnt):
        v = dsb[pl.ds(i * 16, 16)]
        m = (v >= lo) & (v < hi)
        pref = plsc.cumsum(jnp.where(m, 1, 0))
        idx = cnt + pref - 1
        eids = lax.iota(jnp.int32, 16) + (jc * SCH + i * 16)
        plsc.store_scatter(L, [idx], eids, mask=m)
        plsc.store_scatter(Ld, [idx], v, mask=m)
        return jnp.minimum(cnt + pref[15], CAP - 16)

      return lax.fori_loop(0, SCH // 16, inner, cnt)

    cnt = lax.fori_loop(0, NSCH, outer, 0)

    ngr = (cnt + CH - 1) // CH

    def pstep(jc, _):
      pltpu.async_copy(m_hbm.at[L.at[pl.ds(jc * CH, CH)]], mbuf, sem).wait()
      nin = jnp.minimum(CH, cnt - jc * CH)

      def upd(p, _):
        d = Ld[pl.ds(jc * CH + p, 16)][0]
        r = d - lo
        for kk in range(DO // 16):
          sl = pl.ds(kk * 16, 16)
          acc[r, sl] = jnp.maximum(acc[r, sl], mbuf[p, sl])
        return _

      lax.fori_loop(0, nin, upd, 0)
      return _

    lax.fori_loop(0, ngr, pstep, 0)

    def fin(i, _):
      for kk in range(DO // 16):
        sl = pl.ds(kk * 16, 16)
        v = acc[i, sl]
        acc[i, sl] = jnp.where(v == NEG, 0.0, v)
      return _

    lax.fori_loop(0, RNG, fin, 0)
    pltpu.sync_copy(acc, ef_hbm.at[pl.ds(wid * RNG, RNG)])

  return k(dst, M)


def _tc_h(ef, Wc1, bc1, Wc2, bc2):
  RB = 1000
  nb = N // RB

  def body(e_ref, w1_ref, b1_ref, w2_ref, b2_ref, o_ref):
    cmid = jnp.maximum(
        jnp.dot(e_ref[...], w1_ref[...], preferred_element_type=jnp.float32)
        + b1_ref[...], 0.0)
    z = jnp.dot(cmid, w2_ref[...], preferred_element_type=jnp.float32) + b2_ref[...]
    o_ref[...] = jax.nn.sigmoid(z)

  return pl.pallas_call(
      body,
      grid=(nb,),
      in_specs=[
          pl.BlockSpec((RB, DO), lambda i: (i, 0)),
          pl.BlockSpec((DO, DH), lambda i: (0, 0)),
          pl.BlockSpec((1, DH), lambda i: (0, 0)),
          pl.BlockSpec((DH, 1), lambda i: (0, 0)),
          pl.BlockSpec((1, 1), lambda i: (0, 0)),
      ],
      out_specs=pl.BlockSpec((RB, 1), lambda i: (i, 0)),
      out_shape=jax.ShapeDtypeStruct((N, 1), jnp.float32),
  )(ef, Wc1, bc1.reshape(1, DH), Wc2, bc2.reshape(1, 1))


def kernel(x, edge_index, W1, b1, We1, be1, We2, be2, Wc1, bc1, Wc2, bc2):
  src = edge_index[0]
  dst = edge_index[1]
  degp = _sc_hist(dst)
  h, g, dinv = _tc_b(x, W1, degp)
  accp = _sc_scatter_add(src, dst, g)
  P, Q = _tc_d(accp, h, dinv, b1, We1, be1)
  U = _sc_edge_u(src, dst, P, Q)
  M = _tc_f(U, We2, be2)
  efp = _sc_segmax(dst, M)
  out = _tc_h(efp[:N], Wc1, bc1, Wc2, bc2)
  return jnp.squeeze(out, axis=-1)


# speedup vs baseline: 6.4864x; 1.1454x over previous
import functools

import jax
import jax.numpy as jnp
from jax import lax
from jax.experimental import pallas as pl
from jax.experimental.pallas import tpu as pltpu
from jax.experimental.pallas import tpu_sc as plsc

N = 10000
E = 320000
D_IN = 128
DH = 64
DO = 32

NC = 2
NS = 16
NW = NC * NS
EPW = E // NW
CH = 128
NFULL = EPW // CH
TAIL = EPW - NFULL * CH
NP2 = 10240
RSUB = NP2 // NS
ZC = 128
NZC = RSUB // ZC

RNG = 320
NPAD = NW * RNG
SCH = 2000
NSCH = E // SCH
CAP = 16384
NEG = -3.0e38


def _mesh():
  return plsc.VectorSubcoreMesh(core_axis_name="c", subcore_axis_name="s")


_SC_PARAMS = pltpu.CompilerParams(use_tc_tiling_on_sc=False, needs_layout_passes=False)


def _sc_hist(dst):
  @functools.partial(
      pl.kernel,
      mesh=_mesh(),
      compiler_params=_SC_PARAMS,
      out_type=jax.ShapeDtypeStruct((NC, NP2, 16), jnp.float32),
      scratch_types=[
          pltpu.VMEM((CH,), jnp.int32),
          pltpu.VMEM((TAIL,), jnp.int32),
          pltpu.VMEM((CH, 16), jnp.float32),
          pltpu.VMEM((ZC, 16), jnp.float32),
          pltpu.VMEM_SHARED((NP2, 16), jnp.float32),
      ],
  )
  def k(dst_hbm, out_hbm, idxb, idxt, ones, zbuf, hist):
    c = lax.axis_index("c")
    s = lax.axis_index("s")
    wid = s * NC + c
    one = jnp.ones((16,), jnp.float32)
    zero = jnp.zeros((16,), jnp.float32)

    def fill_ones(i, _):
      ones[i, :] = one
      return _

    lax.fori_loop(0, CH, fill_ones, 0)

    def fill_z(i, _):
      zbuf[i, :] = zero
      return _

    lax.fori_loop(0, ZC, fill_z, 0)

    def zc(i, _):
      pltpu.sync_copy(zbuf, hist.at[pl.ds(s * RSUB + i * ZC, ZC)])
      return _

    lax.fori_loop(0, NZC, zc, 0)
    plsc.subcore_barrier()

    base = wid * EPW

    def step(j, _):
      pltpu.sync_copy(dst_hbm.at[pl.ds(base + j * CH, CH)], idxb)
      pltpu.sync_copy(ones, hist.at[idxb], add=True)
      return _

    lax.fori_loop(0, NFULL, step, 0)
    pltpu.sync_copy(dst_hbm.at[pl.ds(base + NFULL * CH, TAIL)], idxt)
    pltpu.sync_copy(ones.at[pl.ds(0, TAIL)], hist.at[idxt], add=True)
    plsc.subcore_barrier()
    pltpu.sync_copy(hist.at[pl.ds(s * RSUB, RSUB)],
                    out_hbm.at[c, pl.ds(s * RSUB, RSUB)])

  return k(dst)


def _tc_b(x, W1, degp):
  RB = 1000
  nb = N // RB

  def body(x_ref, w_ref, d_ref, h_ref, g_ref, dv_ref):
    xx = x_ref[...]
    h = jnp.dot(xx, w_ref[...], preferred_element_type=jnp.float32)
    d = d_ref[...]
    deg = d[0, :, 0] + d[1, :, 0] + 1.0
    dinv = lax.rsqrt(deg)
    h_ref[...] = h
    g_ref[...] = h * dinv[:, None]
    dv_ref[...] = dinv[:, None]

  return pl.pallas_call(
      body,
      grid=(nb,),
      in_specs=[
          pl.BlockSpec((RB, D_IN), lambda i: (i, 0)),
          pl.BlockSpec((D_IN, DH), lambda i: (0, 0)),
          pl.BlockSpec((NC, RB, 16), lambda i: (0, i, 0)),
      ],
      out_specs=[
          pl.BlockSpec((RB, DH), lambda i: (i, 0)),
          pl.BlockSpec((RB, DH), lambda i: (i, 0)),
          pl.BlockSpec((RB, 1), lambda i: (i, 0)),
      ],
      out_shape=[
          jax.ShapeDtypeStruct((N, DH), jnp.float32),
          jax.ShapeDtypeStruct((N, DH), jnp.float32),
          jax.ShapeDtypeStruct((N, 1), jnp.float32),
      ],
  )(x, W1, degp)


def _sc_scatter_add(src, dst, g):
  @functools.partial(
      pl.kernel,
      mesh=_mesh(),
      compiler_params=_SC_PARAMS,
      out_type=jax.ShapeDtypeStruct((NC, NP2, DH), jnp.float32),
      scratch_types=[
          pltpu.VMEM((CH,), jnp.int32),
          pltpu.VMEM((CH,), jnp.int32),
          pltpu.VMEM((CH, DH), jnp.float32),
          pltpu.VMEM((TAIL,), jnp.int32),
          pltpu.VMEM((TAIL,), jnp.int32),
          pltpu.VMEM((TAIL, DH), jnp.float32),
          pltpu.VMEM((ZC, DH), jnp.float32),
          pltpu.SemaphoreType.DMA,
          pltpu.VMEM_SHARED((NP2, DH), jnp.float32),
      ],
  )
  def k(src_hbm, dst_hbm, g_hbm, out_hbm, sbuf, dbuf, gbuf, sbt, dbt, gbt,
        zb, sem, acc):
    c = lax.axis_index("c")
    s = lax.axis_index("s")
    wid = s * NC + c
    zero = jnp.zeros((16,), jnp.float32)

    def zrow(i, _):
      for kk in range(DH // 16):
        zb[i, pl.ds(kk * 16, 16)] = zero
      return _

    lax.fori_loop(0, ZC, zrow, 0)

    def zc(i, _):
      pltpu.sync_copy(zb, acc.at[pl.ds(s * RSUB + i * ZC, ZC)])
      return _

    lax.fori_loop(0, NZC, zc, 0)
    plsc.subcore_barrier()

    base = wid * EPW

    def step(j, _):
      pltpu.sync_copy(src_hbm.at[pl.ds(base + j * CH, CH)], sbuf)
      pltpu.sync_copy(dst_hbm.at[pl.ds(base + j * CH, CH)], dbuf)
      pltpu.async_copy(g_hbm.at[sbuf], gbuf, sem).wait()
      pltpu.sync_copy(gbuf, acc.at[dbuf], add=True)
      return _

    lax.fori_loop(0, NFULL, step, 0)
    pltpu.sync_copy(src_hbm.at[pl.ds(base + NFULL * CH, TAIL)], sbt)
    pltpu.sync_copy(dst_hbm.at[pl.ds(base + NFULL * CH, TAIL)], dbt)
    pltpu.async_copy(g_hbm.at[sbt], gbt, sem).wait()
    pltpu.sync_copy(gbt, acc.at[dbt], add=True)
    plsc.subcore_barrier()

    def dump(i, _):
      pltpu.sync_copy(acc.at[pl.ds(s * RSUB + i * ZC, ZC)],
                      out_hbm.at[c, pl.ds(s * RSUB + i * ZC, ZC)])
      return _

    lax.fori_loop(0, NZC, dump, 0)

  return k(src, dst, g)


def _tc_d(accp, h, dinv, b1, We1, be1):
  RB = 1000
  nb = N // RB

  def body(a_ref, h_ref, dv_ref, b1_ref, w_ref, be_ref, p_ref, q_ref):
    acc = a_ref[0] + a_ref[1]
    dv = dv_ref[...]
    hh = h_ref[...]
    h1 = jnp.maximum(dv * acc + dv * dv * hh + b1_ref[...], 0.0)
    A = w_ref[:DH, :]
    B = w_ref[DH:, :]
    p_ref[...] = jnp.dot(h1, A - B, preferred_element_type=jnp.float32) + be_ref[...]
    q_ref[...] = jnp.dot(h1, B, preferred_element_type=jnp.float32)

  return pl.pallas_call(
      body,
      grid=(nb,),
      in_specs=[
          pl.BlockSpec((NC, RB, DH), lambda i: (0, i, 0)),
          pl.BlockSpec((RB, DH), lambda i: (i, 0)),
          pl.BlockSpec((RB, 1), lambda i: (i, 0)),
          pl.BlockSpec((1, DH), lambda i: (0, 0)),
          pl.BlockSpec((2 * DH, DH), lambda i: (0, 0)),
          pl.BlockSpec((1, DH), lambda i: (0, 0)),
      ],
      out_specs=[
          pl.BlockSpec((RB, DH), lambda i: (i, 0)),
          pl.BlockSpec((RB, DH), lambda i: (i, 0)),
      ],
      out_shape=[
          jax.ShapeDtypeStruct((N, DH), jnp.float32),
          jax.ShapeDtypeStruct((N, DH), jnp.float32),
      ],
  )(accp, h, dinv, b1.reshape(1, DH), We1, be1.reshape(1, DH))


def _sc_edge_u(src, dst, P, Q):
  @functools.partial(
      pl.kernel,
      mesh=_mesh(),
      compiler_params=_SC_PARAMS,
      out_type=jax.ShapeDtypeStruct((E, DH), jnp.float32),
      scratch_types=[
          pltpu.VMEM((CH,), jnp.int32),
          pltpu.VMEM((CH,), jnp.int32),
          pltpu.VMEM((CH, DH), jnp.float32),
          pltpu.VMEM((CH, DH), jnp.float32),
          pltpu.VMEM((TAIL,), jnp.int32),
          pltpu.VMEM((TAIL,), jnp.int32),
          pltpu.VMEM((TAIL, DH), jnp.float32),
          pltpu.VMEM((TAIL, DH), jnp.float32),
          pltpu.SemaphoreType.DMA,
          pltpu.SemaphoreType.DMA,
      ],
  )
  def k(src_hbm, dst_hbm, p_hbm, q_hbm, u_hbm, sbuf, dbuf, pbuf, qbuf,
        sbt, dbt, pbt, qbt, sem1, sem2):
    c = lax.axis_index("c")
    s = lax.axis_index("s")
    wid = s * NC + c
    base = wid * EPW

    def step(j, _):
      pltpu.sync_copy(dst_hbm.at[pl.ds(base + j * CH, CH)], dbuf)
      pltpu.sync_copy(src_hbm.at[pl.ds(base + j * CH, CH)], sbuf)
      cp1 = pltpu.async_copy(p_hbm.at[dbuf], pbuf, sem1)
      cp2 = pltpu.async_copy(q_hbm.at[sbuf], qbuf, sem2)
      cp1.wait()
      cp2.wait()

      def addrow(r, _):
        for kk in range(DH // 16):
          sl = pl.ds(kk * 16, 16)
          pbuf[r, sl] = pbuf[r, sl] + qbuf[r, sl]
        return _

      lax.fori_loop(0, CH, addrow, 0)
      pltpu.sync_copy(pbuf, u_hbm.at[pl.ds(base + j * CH, CH)])
      return _

    lax.fori_loop(0, NFULL, step, 0)

    pltpu.sync_copy(dst_hbm.at[pl.ds(base + NFULL * CH, TAIL)], dbt)
    pltpu.sync_copy(src_hbm.at[pl.ds(base + NFULL * CH, TAIL)], sbt)
    cp1 = pltpu.async_copy(p_hbm.at[dbt], pbt, sem1)
    cp2 = pltpu.async_copy(q_hbm.at[sbt], qbt, sem2)
    cp1.wait()
    cp2.wait()

    def addrow_t(r, _):
      for kk in range(DH // 16):
        sl = pl.ds(kk * 16, 16)
        pbt[r, sl] = pbt[r, sl] + qbt[r, sl]
      return _

    lax.fori_loop(0, TAIL, addrow_t, 0)
    pltpu.sync_copy(pbt, u_hbm.at[pl.ds(base + NFULL * CH, TAIL)])

  return k(src, dst, P, Q)


def _tc_f(U, We2, be2):
  RB = 2000
  nb = E // RB

  def body(u_ref, w_ref, b_ref, m_ref):
    u = jnp.maximum(u_ref[...], 0.0)
    m_ref[...] = jnp.dot(u, w_ref[...], preferred_element_type=jnp.float32) + b_ref[...]

  return pl.pallas_call(
      body,
      grid=(nb,),
      in_specs=[
          pl.BlockSpec((RB, DH), lambda i: (i, 0)),
          pl.BlockSpec((DH, DO), lambda i: (0, 0)),
          pl.BlockSpec((1, DO), lambda i: (0, 0)),
      ],
      out_specs=pl.BlockSpec((RB, DO), lambda i: (i, 0)),
      out_shape=jax.ShapeDtypeStruct((E, DO), jnp.float32),
  )(U, We2, be2.reshape(1, DO))


def _sc_segmax(dst, M):
  @functools.partial(
      pl.kernel,
      mesh=_mesh(),
      compiler_params=_SC_PARAMS,
      out_type=jax.ShapeDtypeStruct((NPAD, DO), jnp.float32),
      scratch_types=[
          pltpu.VMEM((SCH,), jnp.int32),
          pltpu.VMEM((SCH,), jnp.int32),
          pltpu.VMEM((CAP,), jnp.int32),
          pltpu.VMEM((CAP,), jnp.int32),
          pltpu.VMEM((RNG + 8, DO), jnp.float32),
          pltpu.VMEM((CH, DO), jnp.float32),
          pltpu.SemaphoreType.DMA,
          pltpu.SemaphoreType.DMA,
          pltpu.SemaphoreType.DMA,
      ],
  )
  def k(dst_hbm, m_hbm, ef_hbm, dsb0, dsb1, L, Ld, acc, mbuf, sem0, sem1,
        semg):
    c = lax.axis_index("c")
    s = lax.axis_index("s")
    wid = s * NC + c
    lo = wid * RNG
    hi = lo + RNG
    izero = jnp.zeros((16,), jnp.int32)
    negv = jnp.full((16,), NEG, jnp.float32)
    sentv = izero + hi

    def initL(i, _):
      L[pl.ds(i * 16, 16)] = izero
      Ld[pl.ds(i * 16, 16)] = sentv
      return _

    lax.fori_loop(0, CAP // 16, initL, 0)

    def inita(i, _):
      for kk in range(DO // 16):
        acc[i, pl.ds(kk * 16, 16)] = negv
      return _

    lax.fori_loop(0, RNG + 8, inita, 0)

    def scan_buf(buf, cbase, cnt_v):
      def inner(i, cnt_v):
        v = buf[pl.ds(i * 16, 16)]
        m = (v >= lo) & (v < hi)
        pref = plsc.cumsum(jnp.where(m, 1, 0))
        idx = cnt_v + pref - 1
        eids = lax.iota(jnp.int32, 16) + (cbase + i * 16)
        plsc.store_scatter(L, [idx], eids, mask=m)
        plsc.store_scatter(Ld, [idx], v, mask=m)
        popc = plsc.all_reduce_population_count(m)
        return jnp.minimum(cnt_v + popc, CAP - 16)

      return lax.fori_loop(0, SCH // 16, inner, cnt_v)

    pltpu.async_copy(dst_hbm.at[pl.ds(0, SCH)], dsb0, sem0)

    def outer(j2, cnt_v):
      pltpu.async_copy(dst_hbm.at[pl.ds((2 * j2 + 1) * SCH, SCH)], dsb1,
                       sem1)
      pltpu.make_async_copy(dst_hbm.at[pl.ds(0, SCH)], dsb0, sem0).wait()
      cnt_v = scan_buf(dsb0, 2 * j2 * SCH, cnt_v)

      @pl.when(j2 < NSCH // 2 - 1)
      def _():
        pltpu.async_copy(dst_hbm.at[pl.ds((2 * j2 + 2) * SCH, SCH)], dsb0,
                         sem0)

      pltpu.make_async_copy(dst_hbm.at[pl.ds(0, SCH)], dsb1, sem1).wait()
      cnt_v = scan_buf(dsb1, (2 * j2 + 1) * SCH, cnt_v)
      return cnt_v

    cnt_v = lax.fori_loop(0, NSCH // 2, outer, izero)
    cnt = cnt_v[0]

    ngr = (cnt + CH - 1) // CH

    def pstep(jc, _):
      pltpu.async_copy(m_hbm.at[L.at[pl.ds(jc * CH, CH)]], mbuf, semg).wait()

      def grp(q, _):
        dvec = Ld[pl.ds(jc * CH + q * 16, 16)]
        for j in range(16):
          r = dvec[j] - lo
          for kk in range(DO // 16):
            sl = pl.ds(kk * 16, 16)
            acc[r, sl] = jnp.maximum(acc[r, sl], mbuf[q * 16 + j, sl])
        return _

      lax.fori_loop(0, CH // 16, grp, 0)
      return _

    lax.fori_loop(0, ngr, pstep, 0)

    def fin(i, _):
      for kk in range(DO // 16):
        sl = pl.ds(kk * 16, 16)
        v = acc[i, sl]
        acc[i, sl] = jnp.where(v == NEG, 0.0, v)
      return _

    lax.fori_loop(0, RNG, fin, 0)
    pltpu.sync_copy(acc.at[pl.ds(0, RNG)], ef_hbm.at[pl.ds(wid * RNG, RNG)])

  return k(dst, M)


def _tc_h(ef, Wc1, bc1, Wc2, bc2):
  RB = 1000
  nb = N // RB

  def body(e_ref, w1_ref, b1_ref, w2_ref, b2_ref, o_ref):
    cmid = jnp.maximum(
        jnp.dot(e_ref[...], w1_ref[...], preferred_element_type=jnp.float32)
        + b1_ref[...], 0.0)
    z = jnp.dot(cmid, w2_ref[...], preferred_element_type=jnp.float32) + b2_ref[...]
    o_ref[...] = jax.nn.sigmoid(z)

  return pl.pallas_call(
      body,
      grid=(nb,),
      in_specs=[
          pl.BlockSpec((RB, DO), lambda i: (i, 0)),
          pl.BlockSpec((DO, DH), lambda i: (0, 0)),
          pl.BlockSpec((1, DH), lambda i: (0, 0)),
          pl.BlockSpec((DH, 1), lambda i: (0, 0)),
          pl.BlockSpec((1, 1), lambda i: (0, 0)),
      ],
      out_specs=pl.BlockSpec((RB, 1), lambda i: (i, 0)),
      out_shape=jax.ShapeDtypeStruct((N, 1), jnp.float32),
  )(ef, Wc1, bc1.reshape(1, DH), Wc2, bc2.reshape(1, 1))


def kernel(x, edge_index, W1, b1, We1, be1, We2, be2, Wc1, bc1, Wc2, bc2):
  src = edge_index[0]
  dst = edge_index[1]
  degp = _sc_hist(dst)
  h, g, dinv = _tc_b(x, W1, degp)
  accp = _sc_scatter_add(src, dst, g)
  P, Q = _tc_d(accp, h, dinv, b1, We1, be1)
  U = _sc_edge_u(src, dst, P, Q)
  M = _tc_f(U, We2, be2)
  efp = _sc_segmax(dst, M)
  out = _tc_h(efp[:N], Wc1, bc1, Wc2, bc2)
  return jnp.squeeze(out, axis=-1)


# speedup vs baseline: 7.9088x; 1.2193x over previous
import functools

import jax
import jax.numpy as jnp
from jax import lax
from jax.experimental import pallas as pl
from jax.experimental.pallas import tpu as pltpu
from jax.experimental.pallas import tpu_sc as plsc

N = 10000
E = 320000
D_IN = 128
DH = 64
DO = 32

NC = 2
NS = 16
NW = NC * NS
EPW = E // NW
CH = 128
NFULL = EPW // CH
TAIL = EPW - NFULL * CH
NP2 = 10240
RSUB = NP2 // NS
ZC = 128
NZC = RSUB // ZC

RNG = 320
NPAD = NW * RNG
SCH = 2000
NSCH = E // SCH
CAP = 16384
NEG = -3.0e38


def _mesh():
  return plsc.VectorSubcoreMesh(core_axis_name="c", subcore_axis_name="s")


_SC_PARAMS = pltpu.CompilerParams(use_tc_tiling_on_sc=False, needs_layout_passes=False)


def _sc_hist(dst):
  @functools.partial(
      pl.kernel,
      mesh=_mesh(),
      compiler_params=_SC_PARAMS,
      out_type=jax.ShapeDtypeStruct((NC, NP2, 16), jnp.float32),
      scratch_types=[
          pltpu.VMEM((CH,), jnp.int32),
          pltpu.VMEM((TAIL,), jnp.int32),
          pltpu.VMEM((CH, 16), jnp.float32),
          pltpu.VMEM((ZC, 16), jnp.float32),
          pltpu.VMEM_SHARED((NP2, 16), jnp.float32),
      ],
  )
  def k(dst_hbm, out_hbm, idxb, idxt, ones, zbuf, hist):
    c = lax.axis_index("c")
    s = lax.axis_index("s")
    wid = s * NC + c
    one = jnp.ones((16,), jnp.float32)
    zero = jnp.zeros((16,), jnp.float32)

    def fill_ones(i, _):
      ones[i, :] = one
      return _

    lax.fori_loop(0, CH, fill_ones, 0)

    def fill_z(i, _):
      zbuf[i, :] = zero
      return _

    lax.fori_loop(0, ZC, fill_z, 0)

    def zc(i, _):
      pltpu.sync_copy(zbuf, hist.at[pl.ds(s * RSUB + i * ZC, ZC)])
      return _

    lax.fori_loop(0, NZC, zc, 0)
    plsc.subcore_barrier()

    base = wid * EPW

    def step(j, _):
      pltpu.sync_copy(dst_hbm.at[pl.ds(base + j * CH, CH)], idxb)
      pltpu.sync_copy(ones, hist.at[idxb], add=True)
      return _

    lax.fori_loop(0, NFULL, step, 0)
    pltpu.sync_copy(dst_hbm.at[pl.ds(base + NFULL * CH, TAIL)], idxt)
    pltpu.sync_copy(ones.at[pl.ds(0, TAIL)], hist.at[idxt], add=True)
    plsc.subcore_barrier()
    pltpu.sync_copy(hist.at[pl.ds(s * RSUB, RSUB)],
                    out_hbm.at[c, pl.ds(s * RSUB, RSUB)])

  return k(dst)


def _tc_b(x, W1, degp):
  RB = 1000
  nb = N // RB

  def body(x_ref, w_ref, d_ref, h_ref, g_ref, dv_ref):
    xx = x_ref[...]
    h = jnp.dot(xx, w_ref[...], preferred_element_type=jnp.float32)
    d = d_ref[...]
    deg = d[0, :, 0] + d[1, :, 0] + 1.0
    dinv = lax.rsqrt(deg)
    h_ref[...] = h
    g_ref[...] = h * dinv[:, None]
    dv_ref[...] = dinv[:, None]

  return pl.pallas_call(
      body,
      grid=(nb,),
      in_specs=[
          pl.BlockSpec((RB, D_IN), lambda i: (i, 0)),
          pl.BlockSpec((D_IN, DH), lambda i: (0, 0)),
          pl.BlockSpec((NC, RB, 16), lambda i: (0, i, 0)),
      ],
      out_specs=[
          pl.BlockSpec((RB, DH), lambda i: (i, 0)),
          pl.BlockSpec((RB, DH), lambda i: (i, 0)),
          pl.BlockSpec((RB, 1), lambda i: (i, 0)),
      ],
      out_shape=[
          jax.ShapeDtypeStruct((N, DH), jnp.float32),
          jax.ShapeDtypeStruct((N, DH), jnp.float32),
          jax.ShapeDtypeStruct((N, 1), jnp.float32),
      ],
  )(x, W1, degp)


def _sc_scatter_add(src, dst, g):
  @functools.partial(
      pl.kernel,
      mesh=_mesh(),
      compiler_params=_SC_PARAMS,
      out_type=jax.ShapeDtypeStruct((NC, NP2, DH), jnp.float32),
      scratch_types=[
          pltpu.VMEM((EPW,), jnp.int32),
          pltpu.VMEM((CH,), jnp.int32),
          pltpu.VMEM((CH,), jnp.int32),
          pltpu.VMEM((CH, DH), jnp.float32),
          pltpu.VMEM((CH, DH), jnp.float32),
          pltpu.VMEM((TAIL,), jnp.int32),
          pltpu.VMEM((TAIL, DH), jnp.float32),
          pltpu.VMEM((ZC, DH), jnp.float32),
          pltpu.SemaphoreType.DMA,
          pltpu.SemaphoreType.DMA,
          pltpu.SemaphoreType.DMA,
          pltpu.SemaphoreType.DMA,
          pltpu.SemaphoreType.DMA,
          pltpu.VMEM_SHARED((NP2, DH), jnp.float32),
      ],
  )
  def k(src_hbm, dst_hbm, g_hbm, out_hbm, srcall, dbuf0, dbuf1, gbuf0,
        gbuf1, dbt, gbt, zb, semg0, semg1, semi0, semi1, semt, acc):
    c = lax.axis_index("c")
    s = lax.axis_index("s")
    wid = s * NC + c
    zero = jnp.zeros((16,), jnp.float32)

    def zrow(i, _):
      for kk in range(DH // 16):
        zb[i, pl.ds(kk * 16, 16)] = zero
      return _

    lax.fori_loop(0, ZC, zrow, 0)

    def zc(i, _):
      pltpu.sync_copy(zb, acc.at[pl.ds(s * RSUB + i * ZC, ZC)])
      return _

    lax.fori_loop(0, NZC, zc, 0)
    plsc.subcore_barrier()

    base = wid * EPW
    pltpu.sync_copy(src_hbm.at[pl.ds(base, EPW)], srcall)

    def fire(n, dbuf, gbuf, semi, semg):
      pltpu.async_copy(dst_hbm.at[pl.ds(base + n * CH, CH)], dbuf, semi)
      pltpu.async_copy(g_hbm.at[srcall.at[pl.ds(n * CH, CH)]], gbuf, semg)

    def drain(dbuf, gbuf, semi, semg):
      pltpu.make_async_copy(dst_hbm.at[pl.ds(base, CH)], dbuf, semi).wait()
      pltpu.make_async_copy(g_hbm.at[srcall.at[pl.ds(0, CH)]], gbuf,
                            semg).wait()

    fire(0, dbuf0, gbuf0, semi0, semg0)

    def step(j2, carry):
      fire(2 * j2 + 1, dbuf1, gbuf1, semi1, semg1)
      drain(dbuf0, gbuf0, semi0, semg0)
      pltpu.sync_copy(gbuf0, acc.at[dbuf0], add=True)

      @pl.when(j2 < NFULL // 2 - 1)
      def _fn():
        fire(2 * j2 + 2, dbuf0, gbuf0, semi0, semg0)

      drain(dbuf1, gbuf1, semi1, semg1)
      pltpu.sync_copy(gbuf1, acc.at[dbuf1], add=True)
      return carry

    lax.fori_loop(0, NFULL // 2, step, 0)
    pltpu.sync_copy(dst_hbm.at[pl.ds(base + NFULL * CH, TAIL)], dbt)
    pltpu.async_copy(g_hbm.at[srcall.at[pl.ds(NFULL * CH, TAIL)]], gbt,
                     semt).wait()
    pltpu.sync_copy(gbt, acc.at[dbt], add=True)
    plsc.subcore_barrier()

    def dump(i, _):
      pltpu.sync_copy(acc.at[pl.ds(s * RSUB + i * ZC, ZC)],
                      out_hbm.at[c, pl.ds(s * RSUB + i * ZC, ZC)])
      return _

    lax.fori_loop(0, NZC, dump, 0)

  return k(src, dst, g)


def _tc_d(accp, h, dinv, b1, We1, be1):
  RB = 1000
  nb = N // RB

  def body(a_ref, h_ref, dv_ref, b1_ref, w_ref, be_ref, p_ref, q_ref):
    acc = a_ref[0] + a_ref[1]
    dv = dv_ref[...]
    hh = h_ref[...]
    h1 = jnp.maximum(dv * acc + dv * dv * hh + b1_ref[...], 0.0)
    A = w_ref[:DH, :]
    B = w_ref[DH:, :]
    p_ref[...] = jnp.dot(h1, A - B, preferred_element_type=jnp.float32) + be_ref[...]
    q_ref[...] = jnp.dot(h1, B, preferred_element_type=jnp.float32)

  return pl.pallas_call(
      body,
      grid=(nb,),
      in_specs=[
          pl.BlockSpec((NC, RB, DH), lambda i: (0, i, 0)),
          pl.BlockSpec((RB, DH), lambda i: (i, 0)),
          pl.BlockSpec((RB, 1), lambda i: (i, 0)),
          pl.BlockSpec((1, DH), lambda i: (0, 0)),
          pl.BlockSpec((2 * DH, DH), lambda i: (0, 0)),
          pl.BlockSpec((1, DH), lambda i: (0, 0)),
      ],
      out_specs=[
          pl.BlockSpec((RB, DH), lambda i: (i, 0)),
          pl.BlockSpec((RB, DH), lambda i: (i, 0)),
      ],
      out_shape=[
          jax.ShapeDtypeStruct((N, DH), jnp.float32),
          jax.ShapeDtypeStruct((N, DH), jnp.float32),
      ],
  )(accp, h, dinv, b1.reshape(1, DH), We1, be1.reshape(1, DH))


def _sc_edge_u(src, dst, P, Q):
  @functools.partial(
      pl.kernel,
      mesh=_mesh(),
      compiler_params=_SC_PARAMS,
      out_type=jax.ShapeDtypeStruct((E, DH), jnp.float32),
      scratch_types=[
          pltpu.VMEM((EPW,), jnp.int32),
          pltpu.VMEM((EPW,), jnp.int32),
          pltpu.VMEM((CH, DH), jnp.float32),
          pltpu.VMEM((CH, DH), jnp.float32),
          pltpu.VMEM((CH, DH), jnp.float32),
          pltpu.VMEM((CH, DH), jnp.float32),
          pltpu.SemaphoreType.DMA,
          pltpu.SemaphoreType.DMA,
          pltpu.SemaphoreType.DMA,
          pltpu.SemaphoreType.DMA,
      ],
  )
  def k(src_hbm, dst_hbm, p_hbm, q_hbm, u_hbm, srcall, dstall, pbuf0,
        qbuf0, pbuf1, qbuf1, semp0, semq0, semp1, semq1):
    c = lax.axis_index("c")
    s = lax.axis_index("s")
    wid = s * NC + c
    base = wid * EPW
    pltpu.sync_copy(src_hbm.at[pl.ds(base, EPW)], srcall)
    pltpu.sync_copy(dst_hbm.at[pl.ds(base, EPW)], dstall)

    def _bs(buf, nn):
      return buf if nn == CH else buf.at[pl.ds(0, nn)]

    def fire(n, nn, pbuf, qbuf, semp, semq):
      pltpu.async_copy(p_hbm.at[dstall.at[pl.ds(n * CH, nn)]], _bs(pbuf, nn),
                       semp)
      pltpu.async_copy(q_hbm.at[srcall.at[pl.ds(n * CH, nn)]], _bs(qbuf, nn),
                       semq)

    def drain(nn, pbuf, qbuf, semp, semq):
      pltpu.make_async_copy(p_hbm.at[dstall.at[pl.ds(0, nn)]], _bs(pbuf, nn),
                            semp).wait()
      pltpu.make_async_copy(q_hbm.at[srcall.at[pl.ds(0, nn)]], _bs(qbuf, nn),
                            semq).wait()

    def addwrite(n, nn, pbuf, qbuf):
      def addrow(r, _):
        for kk in range(DH // 16):
          sl = pl.ds(kk * 16, 16)
          pbuf[r, sl] = pbuf[r, sl] + qbuf[r, sl]
        return _

      lax.fori_loop(0, nn, addrow, 0)
      pltpu.sync_copy(pbuf if nn == CH else pbuf.at[pl.ds(0, nn)],
                      u_hbm.at[pl.ds(base + n * CH, nn)])

    fire(0, CH, pbuf0, qbuf0, semp0, semq0)

    def step(j2, carry):
      fire(2 * j2 + 1, CH, pbuf1, qbuf1, semp1, semq1)
      drain(CH, pbuf0, qbuf0, semp0, semq0)
      addwrite(2 * j2, CH, pbuf0, qbuf0)

      @pl.when(j2 < NFULL // 2 - 1)
      def _fn():
        fire(2 * j2 + 2, CH, pbuf0, qbuf0, semp0, semq0)

      drain(CH, pbuf1, qbuf1, semp1, semq1)
      addwrite(2 * j2 + 1, CH, pbuf1, qbuf1)
      return carry

    lax.fori_loop(0, NFULL // 2, step, 0)
    fire(NFULL, TAIL, pbuf0, qbuf0, semp0, semq0)
    drain(TAIL, pbuf0, qbuf0, semp0, semq0)
    addwrite(NFULL, TAIL, pbuf0, qbuf0)

  return k(src, dst, P, Q)


def _tc_f(U, We2, be2):
  RB = 2000
  nb = E // RB

  def body(u_ref, w_ref, b_ref, m_ref):
    u = jnp.maximum(u_ref[...], 0.0)
    m_ref[...] = jnp.dot(u, w_ref[...], preferred_element_type=jnp.float32) + b_ref[...]

  return pl.pallas_call(
      body,
      grid=(nb,),
      in_specs=[
          pl.BlockSpec((RB, DH), lambda i: (i, 0)),
          pl.BlockSpec((DH, DO), lambda i: (0, 0)),
          pl.BlockSpec((1, DO), lambda i: (0, 0)),
      ],
      out_specs=pl.BlockSpec((RB, DO), lambda i: (i, 0)),
      out_shape=jax.ShapeDtypeStruct((E, DO), jnp.float32),
  )(U, We2, be2.reshape(1, DO))


def _sc_segmax(dst, M):
  @functools.partial(
      pl.kernel,
      mesh=_mesh(),
      compiler_params=_SC_PARAMS,
      out_type=jax.ShapeDtypeStruct((NPAD, DO), jnp.float32),
      scratch_types=[
          pltpu.VMEM((SCH,), jnp.int32),
          pltpu.VMEM((SCH,), jnp.int32),
          pltpu.VMEM((CAP,), jnp.int32),
          pltpu.VMEM((CAP,), jnp.int32),
          pltpu.VMEM((RNG + 8, DO), jnp.float32),
          pltpu.VMEM((CH, DO), jnp.float32),
          pltpu.SemaphoreType.DMA,
          pltpu.SemaphoreType.DMA,
          pltpu.SemaphoreType.DMA,
      ],
  )
  def k(dst_hbm, m_hbm, ef_hbm, dsb0, dsb1, L, Ld, acc, mbuf, sem0, sem1,
        semg):
    c = lax.axis_index("c")
    s = lax.axis_index("s")
    wid = s * NC + c
    lo = wid * RNG
    hi = lo + RNG
    izero = jnp.zeros((16,), jnp.int32)
    negv = jnp.full((16,), NEG, jnp.float32)
    sentv = izero + hi

    def initL(i, _):
      L[pl.ds(i * 16, 16)] = izero
      Ld[pl.ds(i * 16, 16)] = sentv
      return _

    lax.fori_loop(0, CAP // 16, initL, 0)

    def inita(i, _):
      for kk in range(DO // 16):
        acc[i, pl.ds(kk * 16, 16)] = negv
      return _

    lax.fori_loop(0, RNG + 8, inita, 0)

    def scan_buf(buf, cbase, cnt_v):
      def inner(i, cnt_v):
        v = buf[pl.ds(i * 16, 16)]
        m = (v >= lo) & (v < hi)
        pref = plsc.cumsum(jnp.where(m, 1, 0))
        idx = cnt_v + pref - 1
        eids = lax.iota(jnp.int32, 16) + (cbase + i * 16)
        plsc.store_scatter(L, [idx], eids, mask=m)
        plsc.store_scatter(Ld, [idx], v, mask=m)
        popc = plsc.all_reduce_population_count(m)
        return jnp.minimum(cnt_v + popc, CAP - 16)

      return lax.fori_loop(0, SCH // 16, inner, cnt_v)

    pltpu.async_copy(dst_hbm.at[pl.ds(0, SCH)], dsb0, sem0)

    def outer(j2, cnt_v):
      pltpu.async_copy(dst_hbm.at[pl.ds((2 * j2 + 1) * SCH, SCH)], dsb1,
                       sem1)
      pltpu.make_async_copy(dst_hbm.at[pl.ds(0, SCH)], dsb0, sem0).wait()
      cnt_v = scan_buf(dsb0, 2 * j2 * SCH, cnt_v)

      @pl.when(j2 < NSCH // 2 - 1)
      def _():
        pltpu.async_copy(dst_hbm.at[pl.ds((2 * j2 + 2) * SCH, SCH)], dsb0,
                         sem0)

      pltpu.make_async_copy(dst_hbm.at[pl.ds(0, SCH)], dsb1, sem1).wait()
      cnt_v = scan_buf(dsb1, (2 * j2 + 1) * SCH, cnt_v)
      return cnt_v

    cnt_v = lax.fori_loop(0, NSCH // 2, outer, izero)
    cnt = cnt_v[0]

    ngr = (cnt + CH - 1) // CH

    def pstep(jc, _):
      pltpu.async_copy(m_hbm.at[L.at[pl.ds(jc * CH, CH)]], mbuf, semg).wait()

      def grp(q, _):
        dvec = Ld[pl.ds(jc * CH + q * 16, 16)]
        for j in range(16):
          r = dvec[j] - lo
          for kk in range(DO // 16):
            sl = pl.ds(kk * 16, 16)
            acc[r, sl] = jnp.maximum(acc[r, sl], mbuf[q * 16 + j, sl])
        return _

      lax.fori_loop(0, CH // 16, grp, 0)
      return _

    lax.fori_loop(0, ngr, pstep, 0)

    def fin(i, _):
      for kk in range(DO // 16):
        sl = pl.ds(kk * 16, 16)
        v = acc[i, sl]
        acc[i, sl] = jnp.where(v == NEG, 0.0, v)
      return _

    lax.fori_loop(0, RNG, fin, 0)
    pltpu.sync_copy(acc.at[pl.ds(0, RNG)], ef_hbm.at[pl.ds(wid * RNG, RNG)])

  return k(dst, M)


def _tc_h(ef, Wc1, bc1, Wc2, bc2):
  RB = 1000
  nb = N // RB

  def body(e_ref, w1_ref, b1_ref, w2_ref, b2_ref, o_ref):
    cmid = jnp.maximum(
        jnp.dot(e_ref[...], w1_ref[...], preferred_element_type=jnp.float32)
        + b1_ref[...], 0.0)
    z = jnp.dot(cmid, w2_ref[...], preferred_element_type=jnp.float32) + b2_ref[...]
    o_ref[...] = jax.nn.sigmoid(z)

  return pl.pallas_call(
      body,
      grid=(nb,),
      in_specs=[
          pl.BlockSpec((RB, DO), lambda i: (i, 0)),
          pl.BlockSpec((DO, DH), lambda i: (0, 0)),
          pl.BlockSpec((1, DH), lambda i: (0, 0)),
          pl.BlockSpec((DH, 1), lambda i: (0, 0)),
          pl.BlockSpec((1, 1), lambda i: (0, 0)),
      ],
      out_specs=pl.BlockSpec((RB, 1), lambda i: (i, 0)),
      out_shape=jax.ShapeDtypeStruct((N, 1), jnp.float32),
  )(ef, Wc1, bc1.reshape(1, DH), Wc2, bc2.reshape(1, 1))


def kernel(x, edge_index, W1, b1, We1, be1, We2, be2, Wc1, bc1, Wc2, bc2):
  src = edge_index[0]
  dst = edge_index[1]
  degp = _sc_hist(dst)
  h, g, dinv = _tc_b(x, W1, degp)
  accp = _sc_scatter_add(src, dst, g)
  P, Q = _tc_d(accp, h, dinv, b1, We1, be1)
  U = _sc_edge_u(src, dst, P, Q)
  M = _tc_f(U, We2, be2)
  efp = _sc_segmax(dst, M)
  out = _tc_h(efp[:N], Wc1, bc1, Wc2, bc2)
  return jnp.squeeze(out, axis=-1)


# speedup vs baseline: 9.7873x; 1.2375x over previous
import functools

import jax
import jax.numpy as jnp
from jax import lax
from jax.experimental import pallas as pl
from jax.experimental.pallas import tpu as pltpu
from jax.experimental.pallas import tpu_sc as plsc

N = 10000
E = 320000
D_IN = 128
DH = 64
DO = 32

NC = 2
NS = 16
NW = NC * NS
EPW = E // NW
CH = 128
NFULL = EPW // CH
TAIL = EPW - NFULL * CH
NP2 = 10240
RSUB = NP2 // NS
ZC = 128
NZC = RSUB // ZC

RNG = 320
NB = 32
NPAD = NB * RNG
CAPL = 64
CELLS = NB * 16 * CAPL
DCAP = 8192
NEG = -3.0e38
MASK19 = (1 << 19) - 1


def _mesh():
  return plsc.VectorSubcoreMesh(core_axis_name="c", subcore_axis_name="s")


_SC_PARAMS = pltpu.CompilerParams(use_tc_tiling_on_sc=False, needs_layout_passes=False)


def _sc_hist(dst):
  @functools.partial(
      pl.kernel,
      mesh=_mesh(),
      compiler_params=_SC_PARAMS,
      out_type=jax.ShapeDtypeStruct((NC, NP2, 16), jnp.float32),
      scratch_types=[
          pltpu.VMEM((CH,), jnp.int32),
          pltpu.VMEM((TAIL,), jnp.int32),
          pltpu.VMEM((CH, 16), jnp.float32),
          pltpu.VMEM((ZC, 16), jnp.float32),
          pltpu.VMEM_SHARED((NP2, 16), jnp.float32),
      ],
  )
  def k(dst_hbm, out_hbm, idxb, idxt, ones, zbuf, hist):
    c = lax.axis_index("c")
    s = lax.axis_index("s")
    wid = s * NC + c
    one = jnp.ones((16,), jnp.float32)
    zero = jnp.zeros((16,), jnp.float32)

    def fill_ones(i, _):
      ones[i, :] = one
      return _

    lax.fori_loop(0, CH, fill_ones, 0)

    def fill_z(i, _):
      zbuf[i, :] = zero
      return _

    lax.fori_loop(0, ZC, fill_z, 0)

    def zc(i, _):
      pltpu.sync_copy(zbuf, hist.at[pl.ds(s * RSUB + i * ZC, ZC)])
      return _

    lax.fori_loop(0, NZC, zc, 0)
    plsc.subcore_barrier()

    base = wid * EPW

    def step(j, _):
      pltpu.sync_copy(dst_hbm.at[pl.ds(base + j * CH, CH)], idxb)
      pltpu.sync_copy(ones, hist.at[idxb], add=True)
      return _

    lax.fori_loop(0, NFULL, step, 0)
    pltpu.sync_copy(dst_hbm.at[pl.ds(base + NFULL * CH, TAIL)], idxt)
    pltpu.sync_copy(ones.at[pl.ds(0, TAIL)], hist.at[idxt], add=True)
    plsc.subcore_barrier()
    pltpu.sync_copy(hist.at[pl.ds(s * RSUB, RSUB)],
                    out_hbm.at[c, pl.ds(s * RSUB, RSUB)])

  return k(dst)


def _tc_b(x, W1, degp):
  RB = 1000
  nb = N // RB

  def body(x_ref, w_ref, d_ref, h_ref, g_ref, dv_ref):
    xx = x_ref[...]
    h = jnp.dot(xx, w_ref[...], preferred_element_type=jnp.float32)
    d = d_ref[...]
    deg = d[0, :, 0] + d[1, :, 0] + 1.0
    dinv = lax.rsqrt(deg)
    h_ref[...] = h
    g_ref[...] = h * dinv[:, None]
    dv_ref[...] = dinv[:, None]

  return pl.pallas_call(
      body,
      grid=(nb,),
      in_specs=[
          pl.BlockSpec((RB, D_IN), lambda i: (i, 0)),
          pl.BlockSpec((D_IN, DH), lambda i: (0, 0)),
          pl.BlockSpec((NC, RB, 16), lambda i: (0, i, 0)),
      ],
      out_specs=[
          pl.BlockSpec((RB, DH), lambda i: (i, 0)),
          pl.BlockSpec((RB, DH), lambda i: (i, 0)),
          pl.BlockSpec((RB, 1), lambda i: (i, 0)),
      ],
      out_shape=[
          jax.ShapeDtypeStruct((N, DH), jnp.float32),
          jax.ShapeDtypeStruct((N, DH), jnp.float32),
          jax.ShapeDtypeStruct((N, 1), jnp.float32),
      ],
  )(x, W1, degp)


def _sc_scatter_add(src, dst, g):
  @functools.partial(
      pl.kernel,
      mesh=_mesh(),
      compiler_params=_SC_PARAMS,
      out_type=jax.ShapeDtypeStruct((NC, NP2, DH), jnp.float32),
      scratch_types=[
          pltpu.VMEM((EPW,), jnp.int32),
          pltpu.VMEM((CH,), jnp.int32),
          pltpu.VMEM((CH,), jnp.int32),
          pltpu.VMEM((CH, DH), jnp.float32),
          pltpu.VMEM((CH, DH), jnp.float32),
          pltpu.VMEM((TAIL,), jnp.int32),
          pltpu.VMEM((TAIL, DH), jnp.float32),
          pltpu.VMEM((ZC, DH), jnp.float32),
          pltpu.SemaphoreType.DMA,
          pltpu.SemaphoreType.DMA,
          pltpu.SemaphoreType.DMA,
          pltpu.SemaphoreType.DMA,
          pltpu.SemaphoreType.DMA,
          pltpu.VMEM_SHARED((NP2, DH), jnp.float32),
      ],
  )
  def k(src_hbm, dst_hbm, g_hbm, out_hbm, srcall, dbuf0, dbuf1, gbuf0,
        gbuf1, dbt, gbt, zb, semg0, semg1, semi0, semi1, semt, acc):
    c = lax.axis_index("c")
    s = lax.axis_index("s")
    wid = s * NC + c
    zero = jnp.zeros((16,), jnp.float32)

    def zrow(i, _):
      for kk in range(DH // 16):
        zb[i, pl.ds(kk * 16, 16)] = zero
      return _

    lax.fori_loop(0, ZC, zrow, 0)

    def zc(i, _):
      pltpu.sync_copy(zb, acc.at[pl.ds(s * RSUB + i * ZC, ZC)])
      return _

    lax.fori_loop(0, NZC, zc, 0)
    plsc.subcore_barrier()

    base = wid * EPW
    pltpu.sync_copy(src_hbm.at[pl.ds(base, EPW)], srcall)

    def fire(n, dbuf, gbuf, semi, semg):
      pltpu.async_copy(dst_hbm.at[pl.ds(base + n * CH, CH)], dbuf, semi)
      pltpu.async_copy(g_hbm.at[srcall.at[pl.ds(n * CH, CH)]], gbuf, semg)

    def drain(dbuf, gbuf, semi, semg):
      pltpu.make_async_copy(dst_hbm.at[pl.ds(base, CH)], dbuf, semi).wait()
      pltpu.make_async_copy(g_hbm.at[srcall.at[pl.ds(0, CH)]], gbuf,
                            semg).wait()

    fire(0, dbuf0, gbuf0, semi0, semg0)

    def step(j2, carry):
      fire(2 * j2 + 1, dbuf1, gbuf1, semi1, semg1)
      drain(dbuf0, gbuf0, semi0, semg0)
      pltpu.sync_copy(gbuf0, acc.at[dbuf0], add=True)

      @pl.when(j2 < NFULL // 2 - 1)
      def _fn():
        fire(2 * j2 + 2, dbuf0, gbuf0, semi0, semg0)

      drain(dbuf1, gbuf1, semi1, semg1)
      pltpu.sync_copy(gbuf1, acc.at[dbuf1], add=True)
      return carry

    lax.fori_loop(0, NFULL // 2, step, 0)
    pltpu.sync_copy(dst_hbm.at[pl.ds(base + NFULL * CH, TAIL)], dbt)
    pltpu.async_copy(g_hbm.at[srcall.at[pl.ds(NFULL * CH, TAIL)]], gbt,
                     semt).wait()
    pltpu.sync_copy(gbt, acc.at[dbt], add=True)
    plsc.subcore_barrier()

    def dump(i, _):
      pltpu.sync_copy(acc.at[pl.ds(s * RSUB + i * ZC, ZC)],
                      out_hbm.at[c, pl.ds(s * RSUB + i * ZC, ZC)])
      return _

    lax.fori_loop(0, NZC, dump, 0)

  return k(src, dst, g)


def _tc_d(accp, h, dinv, b1, We1, be1):
  RB = 1000
  nb = N // RB

  def body(a_ref, h_ref, dv_ref, b1_ref, w_ref, be_ref, p_ref, q_ref):
    acc = a_ref[0] + a_ref[1]
    dv = dv_ref[...]
    hh = h_ref[...]
    h1 = jnp.maximum(dv * acc + dv * dv * hh + b1_ref[...], 0.0)
    A = w_ref[:DH, :]
    B = w_ref[DH:, :]
    p_ref[...] = jnp.dot(h1, A - B, preferred_element_type=jnp.float32) + be_ref[...]
    q_ref[...] = jnp.dot(h1, B, preferred_element_type=jnp.float32)

  return pl.pallas_call(
      body,
      grid=(nb,),
      in_specs=[
          pl.BlockSpec((NC, RB, DH), lambda i: (0, i, 0)),
          pl.BlockSpec((RB, DH), lambda i: (i, 0)),
          pl.BlockSpec((RB, 1), lambda i: (i, 0)),
          pl.BlockSpec((1, DH), lambda i: (0, 0)),
          pl.BlockSpec((2 * DH, DH), lambda i: (0, 0)),
          pl.BlockSpec((1, DH), lambda i: (0, 0)),
      ],
      out_specs=[
          pl.BlockSpec((RB, DH), lambda i: (i, 0)),
          pl.BlockSpec((RB, DH), lambda i: (i, 0)),
      ],
      out_shape=[
          jax.ShapeDtypeStruct((N, DH), jnp.float32),
          jax.ShapeDtypeStruct((N, DH), jnp.float32),
      ],
  )(accp, h, dinv, b1.reshape(1, DH), We1, be1.reshape(1, DH))


def _sc_edge_u(src, dst, P, Q):
  @functools.partial(
      pl.kernel,
      mesh=_mesh(),
      compiler_params=_SC_PARAMS,
      out_type=jax.ShapeDtypeStruct((E, DH), jnp.float32),
      scratch_types=[
          pltpu.VMEM((EPW,), jnp.int32),
          pltpu.VMEM((EPW,), jnp.int32),
          pltpu.VMEM((CH, DH), jnp.float32),
          pltpu.VMEM((CH, DH), jnp.float32),
          pltpu.VMEM((CH, DH), jnp.float32),
          pltpu.VMEM((CH, DH), jnp.float32),
          pltpu.SemaphoreType.DMA,
          pltpu.SemaphoreType.DMA,
          pltpu.SemaphoreType.DMA,
          pltpu.SemaphoreType.DMA,
      ],
  )
  def k(src_hbm, dst_hbm, p_hbm, q_hbm, u_hbm, srcall, dstall, pbuf0,
        qbuf0, pbuf1, qbuf1, semp0, semq0, semp1, semq1):
    c = lax.axis_index("c")
    s = lax.axis_index("s")
    wid = s * NC + c
    base = wid * EPW
    pltpu.sync_copy(src_hbm.at[pl.ds(base, EPW)], srcall)
    pltpu.sync_copy(dst_hbm.at[pl.ds(base, EPW)], dstall)

    def _bs(buf, nn):
      return buf if nn == CH else buf.at[pl.ds(0, nn)]

    def fire(n, nn, pbuf, qbuf, semp, semq):
      pltpu.async_copy(p_hbm.at[dstall.at[pl.ds(n * CH, nn)]], _bs(pbuf, nn),
                       semp)
      pltpu.async_copy(q_hbm.at[srcall.at[pl.ds(n * CH, nn)]], _bs(qbuf, nn),
                       semq)

    def drain(nn, pbuf, qbuf, semp, semq):
      pltpu.make_async_copy(p_hbm.at[dstall.at[pl.ds(0, nn)]], _bs(pbuf, nn),
                            semp).wait()
      pltpu.make_async_copy(q_hbm.at[srcall.at[pl.ds(0, nn)]], _bs(qbuf, nn),
                            semq).wait()

    def addwrite(n, nn, pbuf, qbuf):
      def addrow(r, _):
        for kk in range(DH // 16):
          sl = pl.ds(kk * 16, 16)
          pbuf[r, sl] = pbuf[r, sl] + qbuf[r, sl]
        return _

      lax.fori_loop(0, nn, addrow, 0)
      pltpu.sync_copy(pbuf if nn == CH else pbuf.at[pl.ds(0, nn)],
                      u_hbm.at[pl.ds(base + n * CH, nn)])

    fire(0, CH, pbuf0, qbuf0, semp0, semq0)

    def step(j2, carry):
      fire(2 * j2 + 1, CH, pbuf1, qbuf1, semp1, semq1)
      drain(CH, pbuf0, qbuf0, semp0, semq0)
      addwrite(2 * j2, CH, pbuf0, qbuf0)

      @pl.when(j2 < NFULL // 2 - 1)
      def _fn():
        fire(2 * j2 + 2, CH, pbuf0, qbuf0, semp0, semq0)

      drain(CH, pbuf1, qbuf1, semp1, semq1)
      addwrite(2 * j2 + 1, CH, pbuf1, qbuf1)
      return carry

    lax.fori_loop(0, NFULL // 2, step, 0)
    fire(NFULL, TAIL, pbuf0, qbuf0, semp0, semq0)
    drain(TAIL, pbuf0, qbuf0, semp0, semq0)
    addwrite(NFULL, TAIL, pbuf0, qbuf0)

  return k(src, dst, P, Q)


def _tc_f(U, We2, be2):
  RB = 2000
  nb = E // RB

  def body(u_ref, w_ref, b_ref, m_ref):
    u = jnp.maximum(u_ref[...], 0.0)
    m_ref[...] = jnp.dot(u, w_ref[...], preferred_element_type=jnp.float32) + b_ref[...]

  return pl.pallas_call(
      body,
      grid=(nb,),
      in_specs=[
          pl.BlockSpec((RB, DH), lambda i: (i, 0)),
          pl.BlockSpec((DH, DO), lambda i: (0, 0)),
          pl.BlockSpec((1, DO), lambda i: (0, 0)),
      ],
      out_specs=pl.BlockSpec((RB, DO), lambda i: (i, 0)),
      out_shape=jax.ShapeDtypeStruct((E, DO), jnp.float32),
  )(U, We2, be2.reshape(1, DO))


def _sc_segmax(dst, M):
  @functools.partial(
      pl.kernel,
      mesh=_mesh(),
      compiler_params=_SC_PARAMS,
      out_type=jax.ShapeDtypeStruct((NC, NPAD, DO), jnp.float32),
      scratch_types=[
          pltpu.VMEM((EPW,), jnp.int32),
          pltpu.VMEM((CELLS,), jnp.int32),
          pltpu.VMEM((NB * 16,), jnp.int32),
          pltpu.VMEM((16 * CAPL,), jnp.int32),
          pltpu.VMEM((16,), jnp.int32),
          pltpu.VMEM((DCAP,), jnp.int32),
          pltpu.VMEM((DCAP,), jnp.int32),
          pltpu.VMEM((2 * RNG + 16, DO), jnp.float32),
          pltpu.VMEM((CH, DO), jnp.float32),
          pltpu.SemaphoreType.DMA,
          pltpu.VMEM_SHARED((16 * CELLS,), jnp.int32),
          pltpu.VMEM_SHARED((16 * NB * 16,), jnp.int32),
      ],
  )
  def k(dst_hbm, m_hbm, ef_hbm, dstall, cells, cnts, stage, cbuf,
        deid, drow, acc, mbuf, semg, sp_cells, sp_cnts):
    c = lax.axis_index("c")
    s = lax.axis_index("s")
    wid = s * NC + c
    izero = jnp.zeros((16,), jnp.int32)
    negv = jnp.full((16,), NEG, jnp.float32)
    iota = lax.iota(jnp.int32, 16)
    DUMMY = 2 * RNG + 8

    def initc(i, _):
      cnts[pl.ds(i * 16, 16)] = izero
      return _

    lax.fori_loop(0, NB, initc, 0)

    def inita(i, _):
      for kk in range(DO // 16):
        acc[i, pl.ds(kk * 16, 16)] = negv
      return _

    lax.fori_loop(0, 2 * RNG + 16, inita, 0)

    base = wid * EPW
    pltpu.sync_copy(dst_hbm.at[pl.ds(base, EPW)], dstall)

    def scan(i, eids):
      v = dstall[pl.ds(i * 16, 16)]
      b = (v * 13108) >> 22
      doff = v - b * RNG
      penc = (doff << 19) + eids
      cidx = (b << 4) + iota
      cnt = plsc.load_gather(cnts, [cidx])
      pos = cidx * CAPL + jnp.minimum(cnt, CAPL - 1)
      plsc.store_scatter(cells, [pos], penc)
      plsc.store_scatter(cnts, [cidx], cnt + 1)
      return eids + 16

    lax.fori_loop(0, EPW // 16, scan, base + iota)

    pltpu.sync_copy(cells, sp_cells.at[pl.ds(s * CELLS, CELLS)])
    pltpu.sync_copy(cnts, sp_cnts.at[pl.ds(s * (NB * 16), NB * 16)])
    plsc.subcore_barrier()

    for bo in range(2):
      b = 2 * s + bo
      roff = bo * RNG

      def per_worker(s2, dtotal):
        pltpu.sync_copy(
            sp_cells.at[pl.ds(s2 * CELLS + b * (16 * CAPL), 16 * CAPL)],
            stage)
        pltpu.sync_copy(sp_cnts.at[pl.ds(s2 * (NB * 16) + b * 16, 16)], cbuf)
        cvec = cbuf[...]
        for l in range(16):
          cnt = jnp.minimum(cvec[l], CAPL)
          nv = (cnt + 15) >> 4

          def vec(kv, _, l=l, cnt=cnt, dtotal=dtotal):
            off = kv * 16
            w = stage[pl.ds(l * CAPL + off, 16)]
            msk = (off + iota) < cnt
            idx = dtotal + off + iota
            plsc.store_scatter(deid, [idx], w & MASK19, mask=msk)
            plsc.store_scatter(drow, [idx], (w >> 19) + roff, mask=msk)
            return _

          lax.fori_loop(0, nv, vec, 0)
          dtotal = jnp.minimum(dtotal + cnt, DCAP - CH)
        return dtotal

      dtotal = lax.fori_loop(0, 16, per_worker, jnp.int32(0))

      for kv in range(CH // 16):
        idx = dtotal + kv * 16 + iota
        plsc.store_scatter(deid, [idx], izero)
        plsc.store_scatter(drow, [idx], izero + DUMMY)

      ngr = (dtotal + CH - 1) // CH

      def pstep(jc, _):
        pltpu.async_copy(m_hbm.at[deid.at[pl.ds(jc * CH, CH)]], mbuf,
                         semg).wait()

        def grp(q, _):
          dvec = drow[pl.ds(jc * CH + q * 16, 16)]
          for j in range(16):
            r = dvec[j]
            for kk in range(DO // 16):
              sl = pl.ds(kk * 16, 16)
              acc[r, sl] = jnp.maximum(acc[r, sl], mbuf[q * 16 + j, sl])
          return _

        lax.fori_loop(0, CH // 16, grp, 0)
        return _

      lax.fori_loop(0, ngr, pstep, 0)
      pltpu.sync_copy(acc.at[pl.ds(roff, RNG)],
                      ef_hbm.at[c, pl.ds(b * RNG, RNG)])

  return k(dst, M)


def _tc_h(efp, Wc1, bc1, Wc2, bc2):
  RB = 1000
  nb = N // RB

  def body(e_ref, w1_ref, b1_ref, w2_ref, b2_ref, o_ref):
    e = jnp.maximum(e_ref[0], e_ref[1])
    ef = jnp.where(e == NEG, 0.0, e)
    cmid = jnp.maximum(
        jnp.dot(ef, w1_ref[...], preferred_element_type=jnp.float32)
        + b1_ref[...], 0.0)
    z = jnp.dot(cmid, w2_ref[...], preferred_element_type=jnp.float32) + b2_ref[...]
    o_ref[...] = jax.nn.sigmoid(z)

  return pl.pallas_call(
      body,
      grid=(nb,),
      in_specs=[
          pl.BlockSpec((NC, RB, DO), lambda i: (0, i, 0)),
          pl.BlockSpec((DO, DH), lambda i: (0, 0)),
          pl.BlockSpec((1, DH), lambda i: (0, 0)),
          pl.BlockSpec((DH, 1), lambda i: (0, 0)),
          pl.BlockSpec((1, 1), lambda i: (0, 0)),
      ],
      out_specs=pl.BlockSpec((RB, 1), lambda i: (i, 0)),
      out_shape=jax.ShapeDtypeStruct((N, 1), jnp.float32),
  )(efp, Wc1, bc1.reshape(1, DH), Wc2, bc2.reshape(1, 1))


def kernel(x, edge_index, W1, b1, We1, be1, We2, be2, Wc1, bc1, Wc2, bc2):
  src = edge_index[0]
  dst = edge_index[1]
  degp = _sc_hist(dst)
  h, g, dinv = _tc_b(x, W1, degp)
  accp = _sc_scatter_add(src, dst, g)
  P, Q = _tc_d(accp, h, dinv, b1, We1, be1)
  U = _sc_edge_u(src, dst, P, Q)
  M = _tc_f(U, We2, be2)
  efp = _sc_segmax(dst, M)
  out = _tc_h(efp[:, :N], Wc1, bc1, Wc2, bc2)
  return jnp.squeeze(out, axis=-1)


# speedup vs baseline: 13.4878x; 1.3781x over previous
import functools

import jax
import jax.numpy as jnp
from jax import lax
from jax.experimental import pallas as pl
from jax.experimental.pallas import tpu as pltpu
from jax.experimental.pallas import tpu_sc as plsc

N = 10000
E = 320000
D_IN = 128
DH = 64
DO = 32

NC = 2
NS = 16
NW = NC * NS
EPW = E // NW
CH = 128
NFULL = EPW // CH
TAIL = EPW - NFULL * CH
NP2 = 10240
RSUB = NP2 // NS
ZC = 128
NZC = RSUB // ZC

RNG = 320
NB = 32
NPAD = NB * RNG
CAPL = 64
CELLS = NB * 16 * CAPL
DCAP = 8192
NEG = -3.0e38
MASK19 = (1 << 19) - 1


def _mesh():
  return plsc.VectorSubcoreMesh(core_axis_name="c", subcore_axis_name="s")


_SC_PARAMS = pltpu.CompilerParams(use_tc_tiling_on_sc=False, needs_layout_passes=False)


def _sc_hist(dst):
  @functools.partial(
      pl.kernel,
      mesh=_mesh(),
      compiler_params=_SC_PARAMS,
      out_type=jax.ShapeDtypeStruct((NC, NP2, 16), jnp.float32),
      scratch_types=[
          pltpu.VMEM((CH,), jnp.int32),
          pltpu.VMEM((TAIL,), jnp.int32),
          pltpu.VMEM((CH, 16), jnp.float32),
          pltpu.VMEM((ZC, 16), jnp.float32),
          pltpu.VMEM_SHARED((NP2, 16), jnp.float32),
      ],
  )
  def k(dst_hbm, out_hbm, idxb, idxt, ones, zbuf, hist):
    c = lax.axis_index("c")
    s = lax.axis_index("s")
    wid = s * NC + c
    one = jnp.ones((16,), jnp.float32)
    zero = jnp.zeros((16,), jnp.float32)

    def fill_ones(i, _):
      ones[i, :] = one
      return _

    lax.fori_loop(0, CH, fill_ones, 0)

    def fill_z(i, _):
      zbuf[i, :] = zero
      return _

    lax.fori_loop(0, ZC, fill_z, 0)

    def zc(i, _):
      pltpu.sync_copy(zbuf, hist.at[pl.ds(s * RSUB + i * ZC, ZC)])
      return _

    lax.fori_loop(0, NZC, zc, 0)
    plsc.subcore_barrier()

    base = wid * EPW

    def step(j, _):
      pltpu.sync_copy(dst_hbm.at[pl.ds(base + j * CH, CH)], idxb)
      pltpu.sync_copy(ones, hist.at[idxb], add=True)
      return _

    lax.fori_loop(0, NFULL, step, 0)
    pltpu.sync_copy(dst_hbm.at[pl.ds(base + NFULL * CH, TAIL)], idxt)
    pltpu.sync_copy(ones.at[pl.ds(0, TAIL)], hist.at[idxt], add=True)
    plsc.subcore_barrier()
    pltpu.sync_copy(hist.at[pl.ds(s * RSUB, RSUB)],
                    out_hbm.at[c, pl.ds(s * RSUB, RSUB)])

  return k(dst)


def _tc_b(x, W1, degp):
  RB = 1000
  nb = N // RB

  def body(x_ref, w_ref, d_ref, h_ref, g_ref, dv_ref):
    xx = x_ref[...]
    h = jnp.dot(xx, w_ref[...], preferred_element_type=jnp.float32)
    d = d_ref[...]
    deg = d[0, :, 0] + d[1, :, 0] + 1.0
    dinv = lax.rsqrt(deg)
    h_ref[...] = h
    g_ref[...] = h * dinv[:, None]
    dv_ref[...] = dinv[:, None]

  return pl.pallas_call(
      body,
      grid=(nb,),
      in_specs=[
          pl.BlockSpec((RB, D_IN), lambda i: (i, 0)),
          pl.BlockSpec((D_IN, DH), lambda i: (0, 0)),
          pl.BlockSpec((NC, RB, 16), lambda i: (0, i, 0)),
      ],
      out_specs=[
          pl.BlockSpec((RB, DH), lambda i: (i, 0)),
          pl.BlockSpec((RB, DH), lambda i: (i, 0)),
          pl.BlockSpec((RB, 1), lambda i: (i, 0)),
      ],
      out_shape=[
          jax.ShapeDtypeStruct((N, DH), jnp.float32),
          jax.ShapeDtypeStruct((N, DH), jnp.float32),
          jax.ShapeDtypeStruct((N, 1), jnp.float32),
      ],
  )(x, W1, degp)


def _sc_scatter_add(src, dst, g):
  @functools.partial(
      pl.kernel,
      mesh=_mesh(),
      compiler_params=_SC_PARAMS,
      out_type=jax.ShapeDtypeStruct((NC, NP2, DH), jnp.float32),
      scratch_types=[
          pltpu.VMEM((EPW,), jnp.int32),
          pltpu.VMEM((CH,), jnp.int32),
          pltpu.VMEM((CH,), jnp.int32),
          pltpu.VMEM((CH, DH), jnp.float32),
          pltpu.VMEM((CH, DH), jnp.float32),
          pltpu.VMEM((TAIL,), jnp.int32),
          pltpu.VMEM((TAIL, DH), jnp.float32),
          pltpu.VMEM((ZC, DH), jnp.float32),
          pltpu.SemaphoreType.DMA,
          pltpu.SemaphoreType.DMA,
          pltpu.SemaphoreType.DMA,
          pltpu.SemaphoreType.DMA,
          pltpu.SemaphoreType.DMA,
          pltpu.VMEM_SHARED((NP2, DH), jnp.float32),
      ],
  )
  def k(src_hbm, dst_hbm, g_hbm, out_hbm, srcall, dbuf0, dbuf1, gbuf0,
        gbuf1, dbt, gbt, zb, semg0, semg1, semi0, semi1, semt, acc):
    c = lax.axis_index("c")
    s = lax.axis_index("s")
    wid = s * NC + c
    zero = jnp.zeros((16,), jnp.float32)

    def zrow(i, _):
      for kk in range(DH // 16):
        zb[i, pl.ds(kk * 16, 16)] = zero
      return _

    lax.fori_loop(0, ZC, zrow, 0)

    def zc(i, _):
      pltpu.sync_copy(zb, acc.at[pl.ds(s * RSUB + i * ZC, ZC)])
      return _

    lax.fori_loop(0, NZC, zc, 0)
    plsc.subcore_barrier()

    base = wid * EPW
    pltpu.sync_copy(src_hbm.at[pl.ds(base, EPW)], srcall)

    def fire(n, dbuf, gbuf, semi, semg):
      pltpu.async_copy(dst_hbm.at[pl.ds(base + n * CH, CH)], dbuf, semi)
      pltpu.async_copy(g_hbm.at[srcall.at[pl.ds(n * CH, CH)]], gbuf, semg)

    def drain(dbuf, gbuf, semi, semg):
      pltpu.make_async_copy(dst_hbm.at[pl.ds(base, CH)], dbuf, semi).wait()
      pltpu.make_async_copy(g_hbm.at[srcall.at[pl.ds(0, CH)]], gbuf,
                            semg).wait()

    fire(0, dbuf0, gbuf0, semi0, semg0)

    def step(j2, carry):
      fire(2 * j2 + 1, dbuf1, gbuf1, semi1, semg1)
      drain(dbuf0, gbuf0, semi0, semg0)
      pltpu.sync_copy(gbuf0, acc.at[dbuf0], add=True)

      @pl.when(j2 < NFULL // 2 - 1)
      def _fn():
        fire(2 * j2 + 2, dbuf0, gbuf0, semi0, semg0)

      drain(dbuf1, gbuf1, semi1, semg1)
      pltpu.sync_copy(gbuf1, acc.at[dbuf1], add=True)
      return carry

    lax.fori_loop(0, NFULL // 2, step, 0)
    pltpu.sync_copy(dst_hbm.at[pl.ds(base + NFULL * CH, TAIL)], dbt)
    pltpu.async_copy(g_hbm.at[srcall.at[pl.ds(NFULL * CH, TAIL)]], gbt,
                     semt).wait()
    pltpu.sync_copy(gbt, acc.at[dbt], add=True)
    plsc.subcore_barrier()

    def dump(i, _):
      pltpu.sync_copy(acc.at[pl.ds(s * RSUB + i * ZC, ZC)],
                      out_hbm.at[c, pl.ds(s * RSUB + i * ZC, ZC)])
      return _

    lax.fori_loop(0, NZC, dump, 0)

  return k(src, dst, g)


def _tc_d(accp, h, dinv, b1, We1, be1):
  RB = 1000
  nb = N // RB

  def body(a_ref, h_ref, dv_ref, b1_ref, w_ref, be_ref, p_ref, q_ref):
    acc = a_ref[0] + a_ref[1]
    dv = dv_ref[...]
    hh = h_ref[...]
    h1 = jnp.maximum(dv * acc + dv * dv * hh + b1_ref[...], 0.0)
    A = w_ref[:DH, :]
    B = w_ref[DH:, :]
    p_ref[...] = jnp.dot(h1, A - B, preferred_element_type=jnp.float32) + be_ref[...]
    q_ref[...] = jnp.dot(h1, B, preferred_element_type=jnp.float32)

  return pl.pallas_call(
      body,
      grid=(nb,),
      in_specs=[
          pl.BlockSpec((NC, RB, DH), lambda i: (0, i, 0)),
          pl.BlockSpec((RB, DH), lambda i: (i, 0)),
          pl.BlockSpec((RB, 1), lambda i: (i, 0)),
          pl.BlockSpec((1, DH), lambda i: (0, 0)),
          pl.BlockSpec((2 * DH, DH), lambda i: (0, 0)),
          pl.BlockSpec((1, DH), lambda i: (0, 0)),
      ],
      out_specs=[
          pl.BlockSpec((RB, DH), lambda i: (i, 0)),
          pl.BlockSpec((RB, DH), lambda i: (i, 0)),
      ],
      out_shape=[
          jax.ShapeDtypeStruct((N, DH), jnp.float32),
          jax.ShapeDtypeStruct((N, DH), jnp.float32),
      ],
  )(accp, h, dinv, b1.reshape(1, DH), We1, be1.reshape(1, DH))


def _sc_edge_u(src, dst, P, Q):
  @functools.partial(
      pl.kernel,
      mesh=_mesh(),
      compiler_params=_SC_PARAMS,
      out_type=jax.ShapeDtypeStruct((E // 2, 2 * DH), jnp.float32),
      scratch_types=[
          pltpu.VMEM((EPW,), jnp.int32),
          pltpu.VMEM((EPW,), jnp.int32),
          pltpu.VMEM((CH, DH), jnp.float32),
          pltpu.VMEM((CH, DH), jnp.float32),
          pltpu.VMEM((CH, DH), jnp.float32),
          pltpu.VMEM((CH, DH), jnp.float32),
          pltpu.VMEM((CH // 2, 2 * DH), jnp.float32),
          pltpu.SemaphoreType.DMA,
          pltpu.SemaphoreType.DMA,
          pltpu.SemaphoreType.DMA,
          pltpu.SemaphoreType.DMA,
      ],
  )
  def k(src_hbm, dst_hbm, p_hbm, q_hbm, u_hbm, srcall, dstall, pbuf0,
        qbuf0, pbuf1, qbuf1, ustage, semp0, semq0, semp1, semq1):
    c = lax.axis_index("c")
    s = lax.axis_index("s")
    wid = s * NC + c
    base = wid * EPW
    pltpu.sync_copy(src_hbm.at[pl.ds(base, EPW)], srcall)
    pltpu.sync_copy(dst_hbm.at[pl.ds(base, EPW)], dstall)

    def _bs(buf, nn):
      return buf if nn == CH else buf.at[pl.ds(0, nn)]

    def fire(n, nn, pbuf, qbuf, semp, semq):
      pltpu.async_copy(p_hbm.at[dstall.at[pl.ds(n * CH, nn)]], _bs(pbuf, nn),
                       semp)
      pltpu.async_copy(q_hbm.at[srcall.at[pl.ds(n * CH, nn)]], _bs(qbuf, nn),
                       semq)

    def drain(nn, pbuf, qbuf, semp, semq):
      pltpu.make_async_copy(p_hbm.at[dstall.at[pl.ds(0, nn)]], _bs(pbuf, nn),
                            semp).wait()
      pltpu.make_async_copy(q_hbm.at[srcall.at[pl.ds(0, nn)]], _bs(qbuf, nn),
                            semq).wait()

    def addwrite(n, nn, pbuf, qbuf):
      def addrow(r2, _):
        for half in range(2):
          for kk in range(DH // 16):
            sl = pl.ds(kk * 16, 16)
            r = 2 * r2 + half
            ustage[r2, pl.ds(half * DH + kk * 16, 16)] = (
                pbuf[r, sl] + qbuf[r, sl])
        return _

      lax.fori_loop(0, nn // 2, addrow, 0)
      pltpu.sync_copy(ustage if nn == CH else ustage.at[pl.ds(0, nn // 2)],
                      u_hbm.at[pl.ds((base + n * CH) // 2, nn // 2)])

    fire(0, CH, pbuf0, qbuf0, semp0, semq0)

    def step(j2, carry):
      fire(2 * j2 + 1, CH, pbuf1, qbuf1, semp1, semq1)
      drain(CH, pbuf0, qbuf0, semp0, semq0)
      addwrite(2 * j2, CH, pbuf0, qbuf0)

      @pl.when(j2 < NFULL // 2 - 1)
      def _fn():
        fire(2 * j2 + 2, CH, pbuf0, qbuf0, semp0, semq0)

      drain(CH, pbuf1, qbuf1, semp1, semq1)
      addwrite(2 * j2 + 1, CH, pbuf1, qbuf1)
      return carry

    lax.fori_loop(0, NFULL // 2, step, 0)
    fire(NFULL, TAIL, pbuf0, qbuf0, semp0, semq0)
    drain(TAIL, pbuf0, qbuf0, semp0, semq0)
    addwrite(NFULL, TAIL, pbuf0, qbuf0)

  return k(src, dst, P, Q)


def _tc_f(U2, We2, be2):
  RBP = 800
  nb = (E // 4) // RBP

  W4 = jnp.zeros((2 * DH, 2 * DO), jnp.float32)
  W4 = W4.at[:DH, :DO].set(We2).at[DH:, DO:].set(We2)
  b4 = jnp.concatenate([be2, be2]).reshape(1, 2 * DO)

  def body(ua_ref, ub_ref, w_ref, b_ref, m_ref):
    w = w_ref[...]
    b = b_ref[...]
    mL = jnp.dot(jnp.maximum(ua_ref[...], 0.0), w,
                 preferred_element_type=jnp.float32) + b
    mR = jnp.dot(jnp.maximum(ub_ref[...], 0.0), w,
                 preferred_element_type=jnp.float32) + b
    m_ref[...] = jnp.concatenate([mL, mR], axis=1)

  return pl.pallas_call(
      body,
      grid=(nb,),
      in_specs=[
          pl.BlockSpec((RBP, 2 * DH), lambda i: (i, 0)),
          pl.BlockSpec((RBP, 2 * DH), lambda i: (i + nb, 0)),
          pl.BlockSpec((2 * DH, 2 * DO), lambda i: (0, 0)),
          pl.BlockSpec((1, 2 * DO), lambda i: (0, 0)),
      ],
      out_specs=pl.BlockSpec((RBP, 4 * DO), lambda i: (i, 0)),
      out_shape=jax.ShapeDtypeStruct((E // 4, 4 * DO), jnp.float32),
  )(U2, U2, W4, b4)


def _sc_segmax(dst, M):
  @functools.partial(
      pl.kernel,
      mesh=_mesh(),
      compiler_params=_SC_PARAMS,
      out_type=jax.ShapeDtypeStruct((NC, NPAD, DO), jnp.float32),
      scratch_types=[
          pltpu.VMEM((EPW,), jnp.int32),
          pltpu.VMEM((CELLS,), jnp.int32),
          pltpu.VMEM((NB * 16,), jnp.int32),
          pltpu.VMEM((16 * CAPL,), jnp.int32),
          pltpu.VMEM((16,), jnp.int32),
          pltpu.VMEM((DCAP,), jnp.int32),
          pltpu.VMEM((DCAP,), jnp.int32),
          pltpu.VMEM((2 * RNG + 16, DO), jnp.float32),
          pltpu.VMEM((CH, 2 * DO), jnp.float32),
          pltpu.SemaphoreType.DMA,
          pltpu.VMEM_SHARED((16 * CELLS,), jnp.int32),
          pltpu.VMEM_SHARED((16 * NB * 16,), jnp.int32),
      ],
  )
  def k(dst_hbm, m_hbm, ef_hbm, dstall, cells, cnts, stage, cbuf,
        deid, drow, acc, mbuf, semg, sp_cells, sp_cnts):
    c = lax.axis_index("c")
    s = lax.axis_index("s")
    wid = s * NC + c
    izero = jnp.zeros((16,), jnp.int32)
    negv = jnp.full((16,), NEG, jnp.float32)
    iota = lax.iota(jnp.int32, 16)
    DUMMY = 2 * RNG + 8

    def initc(i, _):
      cnts[pl.ds(i * 16, 16)] = izero
      return _

    lax.fori_loop(0, NB, initc, 0)

    def inita(i, _):
      for kk in range(DO // 16):
        acc[i, pl.ds(kk * 16, 16)] = negv
      return _

    lax.fori_loop(0, 2 * RNG + 16, inita, 0)

    base = wid * EPW
    pltpu.sync_copy(dst_hbm.at[pl.ds(base, EPW)], dstall)

    def scan(i, eids):
      v = dstall[pl.ds(i * 16, 16)]
      b = (v * 13108) >> 22
      doff = v - b * RNG
      penc = (doff << 19) + eids
      cidx = (b << 4) + iota
      cnt = plsc.load_gather(cnts, [cidx])
      pos = cidx * CAPL + jnp.minimum(cnt, CAPL - 1)
      plsc.store_scatter(cells, [pos], penc)
      plsc.store_scatter(cnts, [cidx], cnt + 1)
      return eids + 16

    lax.fori_loop(0, EPW // 16, scan, base + iota)

    pltpu.sync_copy(cells, sp_cells.at[pl.ds(s * CELLS, CELLS)])
    pltpu.sync_copy(cnts, sp_cnts.at[pl.ds(s * (NB * 16), NB * 16)])
    plsc.subcore_barrier()

    for bo in range(2):
      b = 2 * s + bo
      roff = bo * RNG

      def per_worker(s2, dtotal):
        pltpu.sync_copy(
            sp_cells.at[pl.ds(s2 * CELLS + b * (16 * CAPL), 16 * CAPL)],
            stage)
        pltpu.sync_copy(sp_cnts.at[pl.ds(s2 * (NB * 16) + b * 16, 16)], cbuf)
        cvec = cbuf[...]
        for l in range(16):
          cnt = jnp.minimum(cvec[l], CAPL)
          nv = (cnt + 15) >> 4

          def vec(kv, _, l=l, cnt=cnt, dtotal=dtotal):
            off = kv * 16
            w = stage[pl.ds(l * CAPL + off, 16)]
            msk = (off + iota) < cnt
            idx = dtotal + off + iota
            ev = w & MASK19
            p = ev >> 1
            row = jnp.where(p < E // 4, p << 1, ((p - E // 4) << 1) + 1)
            plsc.store_scatter(deid, [idx], row, mask=msk)
            plsc.store_scatter(drow, [idx],
                              (w >> 19) + roff + ((ev & 1) << 12), mask=msk)
            return _

          lax.fori_loop(0, nv, vec, 0)
          dtotal = jnp.minimum(dtotal + cnt, DCAP - CH)
        return dtotal

      dtotal = lax.fori_loop(0, 16, per_worker, jnp.int32(0))

      for kv in range(CH // 16):
        idx = dtotal + kv * 16 + iota
        plsc.store_scatter(deid, [idx], izero)
        plsc.store_scatter(drow, [idx], izero + DUMMY)

      ngr = (dtotal + CH - 1) // CH

      def pstep(jc, _):
        pltpu.async_copy(m_hbm.at[deid.at[pl.ds(jc * CH, CH)]], mbuf,
                         semg).wait()

        def grp(q, _):
          dvec = drow[pl.ds(jc * CH + q * 16, 16)]
          for j in range(16):
            dvj = dvec[j]
            r = dvj & 4095
            par = dvj >> 12
            for kk in range(DO // 16):
              sl = pl.ds(kk * 16, 16)
              msl = pl.ds(par * DO + kk * 16, 16)
              acc[r, sl] = jnp.maximum(acc[r, sl], mbuf[q * 16 + j, msl])
          return _

        lax.fori_loop(0, CH // 16, grp, 0)
        return _

      lax.fori_loop(0, ngr, pstep, 0)
      pltpu.sync_copy(acc.at[pl.ds(roff, RNG)],
                      ef_hbm.at[c, pl.ds(b * RNG, RNG)])

  return k(dst, M)


def _tc_h(efp, Wc1, bc1, Wc2, bc2):
  RB = 1000
  nb = N // RB

  def body(e_ref, w1_ref, b1_ref, w2_ref, b2_ref, o_ref):
    e = jnp.maximum(e_ref[0], e_ref[1])
    ef = jnp.where(e == NEG, 0.0, e)
    cmid = jnp.maximum(
        jnp.dot(ef, w1_ref[...], preferred_element_type=jnp.float32)
        + b1_ref[...], 0.0)
    z = jnp.dot(cmid, w2_ref[...], preferred_element_type=jnp.float32) + b2_ref[...]
    o_ref[...] = jax.nn.sigmoid(z)

  return pl.pallas_call(
      body,
      grid=(nb,),
      in_specs=[
          pl.BlockSpec((NC, RB, DO), lambda i: (0, i, 0)),
          pl.BlockSpec((DO, DH), lambda i: (0, 0)),
          pl.BlockSpec((1, DH), lambda i: (0, 0)),
          pl.BlockSpec((DH, 1), lambda i: (0, 0)),
          pl.BlockSpec((1, 1), lambda i: (0, 0)),
      ],
      out_specs=pl.BlockSpec((RB, 1), lambda i: (i, 0)),
      out_shape=jax.ShapeDtypeStruct((N, 1), jnp.float32),
  )(efp, Wc1, bc1.reshape(1, DH), Wc2, bc2.reshape(1, 1))


def kernel(x, edge_index, W1, b1, We1, be1, We2, be2, Wc1, bc1, Wc2, bc2):
  src = edge_index[0]
  dst = edge_index[1]
  degp = _sc_hist(dst)
  h, g, dinv = _tc_b(x, W1, degp)
  accp = _sc_scatter_add(src, dst, g)
  P, Q = _tc_d(accp, h, dinv, b1, We1, be1)
  U2 = _sc_edge_u(src, dst, P, Q)
  M4 = _tc_f(U2, We2, be2)
  efp = _sc_segmax(dst, M4.reshape(E // 2, 2 * DO))
  out = _tc_h(efp[:, :N], Wc1, bc1, Wc2, bc2)
  return jnp.squeeze(out, axis=-1)


# speedup vs baseline: 16.4494x; 1.2196x over previous
import functools

import jax
import jax.numpy as jnp
from jax import lax
from jax.experimental import pallas as pl
from jax.experimental.pallas import tpu as pltpu
from jax.experimental.pallas import tpu_sc as plsc

N = 10000
E = 320000
D_IN = 128
DH = 64
DO = 32

NC = 2
NS = 16
NW = NC * NS
EPW = E // NW
CH = 128
NFULL = EPW // CH
TAIL = EPW - NFULL * CH
NP2 = 10240
RSUB = NP2 // NS
ZC = 128
NZC = RSUB // ZC

RNG = 320
NB = 32
NPAD = NB * RNG
CAPL = 64
CELLS = NB * 16 * CAPL
DCAP = 8192
GCH = 64
NEG = -3.0e38
MASK19 = (1 << 19) - 1


def _mesh():
  return plsc.VectorSubcoreMesh(core_axis_name="c", subcore_axis_name="s")


_SC_PARAMS = pltpu.CompilerParams(use_tc_tiling_on_sc=False, needs_layout_passes=False)


def _sc_hist(ei):
  @functools.partial(
      pl.kernel,
      mesh=_mesh(),
      compiler_params=_SC_PARAMS,
      out_type=jax.ShapeDtypeStruct((NC, NP2, 16), jnp.float32),
      scratch_types=[
          pltpu.VMEM((CH,), jnp.int32),
          pltpu.VMEM((TAIL,), jnp.int32),
          pltpu.VMEM((CH, 16), jnp.float32),
          pltpu.VMEM((ZC, 16), jnp.float32),
          pltpu.VMEM_SHARED((NP2, 16), jnp.float32),
      ],
  )
  def k(ei_hbm, out_hbm, idxb, idxt, ones, zbuf, hist):
    c = lax.axis_index("c")
    s = lax.axis_index("s")
    wid = s * NC + c
    one = jnp.ones((16,), jnp.float32)
    zero = jnp.zeros((16,), jnp.float32)

    def fill_ones(i, _):
      ones[i, :] = one
      return _

    lax.fori_loop(0, CH, fill_ones, 0)

    def fill_z(i, _):
      zbuf[i, :] = zero
      return _

    lax.fori_loop(0, ZC, fill_z, 0)

    def zc(i, _):
      pltpu.sync_copy(zbuf, hist.at[pl.ds(s * RSUB + i * ZC, ZC)])
      return _

    lax.fori_loop(0, NZC, zc, 0)
    plsc.subcore_barrier()

    base = wid * EPW

    def step(j, _):
      pltpu.sync_copy(ei_hbm.at[1, pl.ds(base + j * CH, CH)], idxb)
      pltpu.sync_copy(ones, hist.at[idxb], add=True)
      return _

    lax.fori_loop(0, NFULL, step, 0)
    pltpu.sync_copy(ei_hbm.at[1, pl.ds(base + NFULL * CH, TAIL)], idxt)
    pltpu.sync_copy(ones.at[pl.ds(0, TAIL)], hist.at[idxt], add=True)
    plsc.subcore_barrier()
    pltpu.sync_copy(hist.at[pl.ds(s * RSUB, RSUB)],
                    out_hbm.at[c, pl.ds(s * RSUB, RSUB)])

  return k(ei)


def _tc_b(x, W1, degp):
  RB = 1000
  nb = N // RB

  def body(x_ref, w_ref, d_ref, h_ref, g_ref, dv_ref):
    xx = x_ref[...]
    h = jnp.dot(xx, w_ref[...], preferred_element_type=jnp.float32)
    d = d_ref[...]
    deg = d[0, :, 0] + d[1, :, 0] + 1.0
    dinv = lax.rsqrt(deg)
    h_ref[...] = h
    g_ref[...] = h * dinv[:, None]
    dv_ref[...] = dinv[:, None]

  return pl.pallas_call(
      body,
      grid=(nb,),
      in_specs=[
          pl.BlockSpec((RB, D_IN), lambda i: (i, 0)),
          pl.BlockSpec((D_IN, DH), lambda i: (0, 0)),
          pl.BlockSpec((NC, RB, 16), lambda i: (0, i, 0)),
      ],
      out_specs=[
          pl.BlockSpec((RB, DH), lambda i: (i, 0)),
          pl.BlockSpec((RB, DH), lambda i: (i, 0)),
          pl.BlockSpec((RB, 1), lambda i: (i, 0)),
      ],
      out_shape=[
          jax.ShapeDtypeStruct((N, DH), jnp.float32),
          jax.ShapeDtypeStruct((N, DH), jnp.float32),
          jax.ShapeDtypeStruct((N, 1), jnp.float32),
      ],
  )(x, W1, degp)


def _sc_scatter_add(ei, g):
  @functools.partial(
      pl.kernel,
      mesh=_mesh(),
      compiler_params=_SC_PARAMS,
      out_type=jax.ShapeDtypeStruct((NC, NP2, DH), jnp.float32),
      scratch_types=[
          pltpu.VMEM((EPW,), jnp.int32),
          pltpu.VMEM((CH,), jnp.int32),
          pltpu.VMEM((CH,), jnp.int32),
          pltpu.VMEM((CH, DH), jnp.float32),
          pltpu.VMEM((CH, DH), jnp.float32),
          pltpu.VMEM((TAIL,), jnp.int32),
          pltpu.VMEM((TAIL, DH), jnp.float32),
          pltpu.VMEM((ZC, DH), jnp.float32),
          pltpu.SemaphoreType.DMA,
          pltpu.SemaphoreType.DMA,
          pltpu.SemaphoreType.DMA,
          pltpu.SemaphoreType.DMA,
          pltpu.SemaphoreType.DMA,
          pltpu.VMEM_SHARED((NP2, DH), jnp.float32),
      ],
  )
  def k(ei_hbm, g_hbm, out_hbm, srcall, dbuf0, dbuf1, gbuf0,
        gbuf1, dbt, gbt, zb, semg0, semg1, semi0, semi1, semt, acc):
    c = lax.axis_index("c")
    s = lax.axis_index("s")
    wid = s * NC + c
    zero = jnp.zeros((16,), jnp.float32)

    def zrow(i, _):
      for kk in range(DH // 16):
        zb[i, pl.ds(kk * 16, 16)] = zero
      return _

    lax.fori_loop(0, ZC, zrow, 0)

    def zc(i, _):
      pltpu.sync_copy(zb, acc.at[pl.ds(s * RSUB + i * ZC, ZC)])
      return _

    lax.fori_loop(0, NZC, zc, 0)
    plsc.subcore_barrier()

    base = wid * EPW
    pltpu.sync_copy(ei_hbm.at[0, pl.ds(base, EPW)], srcall)

    def fire(n, dbuf, gbuf, semi, semg):
      pltpu.async_copy(ei_hbm.at[1, pl.ds(base + n * CH, CH)], dbuf, semi)
      pltpu.async_copy(g_hbm.at[srcall.at[pl.ds(n * CH, CH)]], gbuf, semg)

    def drain(dbuf, gbuf, semi, semg):
      pltpu.make_async_copy(ei_hbm.at[1, pl.ds(base, CH)], dbuf, semi).wait()
      pltpu.make_async_copy(g_hbm.at[srcall.at[pl.ds(0, CH)]], gbuf,
                            semg).wait()

    fire(0, dbuf0, gbuf0, semi0, semg0)

    def step(j2, carry):
      fire(2 * j2 + 1, dbuf1, gbuf1, semi1, semg1)
      drain(dbuf0, gbuf0, semi0, semg0)
      pltpu.sync_copy(gbuf0, acc.at[dbuf0], add=True)

      @pl.when(j2 < NFULL // 2 - 1)
      def _fn():
        fire(2 * j2 + 2, dbuf0, gbuf0, semi0, semg0)

      drain(dbuf1, gbuf1, semi1, semg1)
      pltpu.sync_copy(gbuf1, acc.at[dbuf1], add=True)
      return carry

    lax.fori_loop(0, NFULL // 2, step, 0)
    pltpu.sync_copy(ei_hbm.at[1, pl.ds(base + NFULL * CH, TAIL)], dbt)
    pltpu.async_copy(g_hbm.at[srcall.at[pl.ds(NFULL * CH, TAIL)]], gbt,
                     semt).wait()
    pltpu.sync_copy(gbt, acc.at[dbt], add=True)
    plsc.subcore_barrier()

    def dump(i, _):
      pltpu.sync_copy(acc.at[pl.ds(s * RSUB + i * ZC, ZC)],
                      out_hbm.at[c, pl.ds(s * RSUB + i * ZC, ZC)])
      return _

    lax.fori_loop(0, NZC, dump, 0)

  return k(ei, g)


def _tc_d(accp, h, dinv, b1, We1, be1):
  RB = 1000
  nb = N // RB

  def body(a_ref, h_ref, dv_ref, b1_ref, w_ref, be_ref, p_ref, q_ref):
    acc = a_ref[0] + a_ref[1]
    dv = dv_ref[...]
    hh = h_ref[...]
    h1 = jnp.maximum(dv * acc + dv * dv * hh + b1_ref[...], 0.0)
    A = w_ref[:DH, :]
    B = w_ref[DH:, :]
    p_ref[...] = jnp.dot(h1, A - B, preferred_element_type=jnp.float32) + be_ref[...]
    q_ref[...] = jnp.dot(h1, B, preferred_element_type=jnp.float32)

  return pl.pallas_call(
      body,
      grid=(nb,),
      in_specs=[
          pl.BlockSpec((NC, RB, DH), lambda i: (0, i, 0)),
          pl.BlockSpec((RB, DH), lambda i: (i, 0)),
          pl.BlockSpec((RB, 1), lambda i: (i, 0)),
          pl.BlockSpec((1, DH), lambda i: (0, 0)),
          pl.BlockSpec((2 * DH, DH), lambda i: (0, 0)),
          pl.BlockSpec((1, DH), lambda i: (0, 0)),
      ],
      out_specs=[
          pl.BlockSpec((RB, DH), lambda i: (i, 0)),
          pl.BlockSpec((RB, DH), lambda i: (i, 0)),
      ],
      out_shape=[
          jax.ShapeDtypeStruct((N, DH), jnp.float32),
          jax.ShapeDtypeStruct((N, DH), jnp.float32),
      ],
  )(accp, h, dinv, b1.reshape(1, DH), We1, be1.reshape(1, DH))


def _sc_edge_u(ei, P, Q):
  @functools.partial(
      pl.kernel,
      mesh=_mesh(),
      compiler_params=_SC_PARAMS,
      out_type=jax.ShapeDtypeStruct((E // 2, 2 * DH), jnp.float32),
      scratch_types=[
          pltpu.VMEM((EPW,), jnp.int32),
          pltpu.VMEM((EPW,), jnp.int32),
          pltpu.VMEM((CH, DH), jnp.float32),
          pltpu.VMEM((CH, DH), jnp.float32),
          pltpu.VMEM((CH, DH), jnp.float32),
          pltpu.VMEM((CH, DH), jnp.float32),
          pltpu.VMEM((CH // 2, 2 * DH), jnp.float32),
          pltpu.SemaphoreType.DMA,
          pltpu.SemaphoreType.DMA,
          pltpu.SemaphoreType.DMA,
          pltpu.SemaphoreType.DMA,
      ],
  )
  def k(ei_hbm, p_hbm, q_hbm, u_hbm, srcall, dstall, pbuf0,
        qbuf0, pbuf1, qbuf1, ustage, semp0, semq0, semp1, semq1):
    c = lax.axis_index("c")
    s = lax.axis_index("s")
    wid = s * NC + c
    base = wid * EPW
    pltpu.sync_copy(ei_hbm.at[0, pl.ds(base, EPW)], srcall)
    pltpu.sync_copy(ei_hbm.at[1, pl.ds(base, EPW)], dstall)

    def _bs(buf, nn):
      return buf if nn == CH else buf.at[pl.ds(0, nn)]

    def fire(n, nn, pbuf, qbuf, semp, semq):
      pltpu.async_copy(p_hbm.at[dstall.at[pl.ds(n * CH, nn)]], _bs(pbuf, nn),
                       semp)
      pltpu.async_copy(q_hbm.at[srcall.at[pl.ds(n * CH, nn)]], _bs(qbuf, nn),
                       semq)

    def drain(nn, pbuf, qbuf, semp, semq):
      pltpu.make_async_copy(p_hbm.at[dstall.at[pl.ds(0, nn)]], _bs(pbuf, nn),
                            semp).wait()
      pltpu.make_async_copy(q_hbm.at[srcall.at[pl.ds(0, nn)]], _bs(qbuf, nn),
                            semq).wait()

    def addwrite(n, nn, pbuf, qbuf):
      def addrow(r2, _):
        for half in range(2):
          for kk in range(DH // 16):
            sl = pl.ds(kk * 16, 16)
            r = 2 * r2 + half
            ustage[r2, pl.ds(half * DH + kk * 16, 16)] = (
                pbuf[r, sl] + qbuf[r, sl])
        return _

      lax.fori_loop(0, nn // 2, addrow, 0)
      pltpu.sync_copy(ustage if nn == CH else ustage.at[pl.ds(0, nn // 2)],
                      u_hbm.at[pl.ds((base + n * CH) // 2, nn // 2)])

    fire(0, CH, pbuf0, qbuf0, semp0, semq0)

    def step(j2, carry):
      fire(2 * j2 + 1, CH, pbuf1, qbuf1, semp1, semq1)
      drain(CH, pbuf0, qbuf0, semp0, semq0)
      addwrite(2 * j2, CH, pbuf0, qbuf0)

      @pl.when(j2 < NFULL // 2 - 1)
      def _fn():
        fire(2 * j2 + 2, CH, pbuf0, qbuf0, semp0, semq0)

      drain(CH, pbuf1, qbuf1, semp1, semq1)
      addwrite(2 * j2 + 1, CH, pbuf1, qbuf1)
      return carry

    lax.fori_loop(0, NFULL // 2, step, 0)
    fire(NFULL, TAIL, pbuf0, qbuf0, semp0, semq0)
    drain(TAIL, pbuf0, qbuf0, semp0, semq0)
    addwrite(NFULL, TAIL, pbuf0, qbuf0)

  return k(ei, P, Q)


def _tc_f(U2, We2, be2):
  RBP = 1600
  nb = (E // 4) // RBP

  W4 = jnp.zeros((2 * DH, 2 * DO), jnp.float32)
  W4 = W4.at[:DH, :DO].set(We2).at[DH:, DO:].set(We2)
  b4 = jnp.concatenate([be2, be2]).reshape(1, 2 * DO)

  def body(ua_ref, ub_ref, w_ref, b_ref, m_ref):
    w = w_ref[...]
    b = b_ref[...]
    mL = jnp.dot(jnp.maximum(ua_ref[...], 0.0), w,
                 preferred_element_type=jnp.float32) + b
    mR = jnp.dot(jnp.maximum(ub_ref[...], 0.0), w,
                 preferred_element_type=jnp.float32) + b
    m_ref[...] = jnp.concatenate([mL, mR], axis=1)

  return pl.pallas_call(
      body,
      grid=(nb,),
      in_specs=[
          pl.BlockSpec((RBP, 2 * DH), lambda i: (i, 0)),
          pl.BlockSpec((RBP, 2 * DH), lambda i: (i + nb, 0)),
          pl.BlockSpec((2 * DH, 2 * DO), lambda i: (0, 0)),
          pl.BlockSpec((1, 2 * DO), lambda i: (0, 0)),
      ],
      out_specs=pl.BlockSpec((RBP, 4 * DO), lambda i: (i, 0)),
      out_shape=jax.ShapeDtypeStruct((E // 4, 4 * DO), jnp.float32),
  )(U2, U2, W4, b4)


def _sc_segmax(dst, M):
  @functools.partial(
      pl.kernel,
      mesh=_mesh(),
      compiler_params=_SC_PARAMS,
      out_type=jax.ShapeDtypeStruct((NC, NPAD, DO), jnp.float32),
      scratch_types=[
          pltpu.VMEM((EPW,), jnp.int32),
          pltpu.VMEM((CELLS,), jnp.int32),
          pltpu.VMEM((NB * 16,), jnp.int32),
          pltpu.VMEM((16 * CAPL,), jnp.int32),
          pltpu.VMEM((16,), jnp.int32),
          pltpu.VMEM((DCAP,), jnp.int32),
          pltpu.VMEM((DCAP,), jnp.int32),
          pltpu.VMEM((2 * RNG + 16, DO), jnp.float32),
          pltpu.VMEM((GCH, 2 * DO), jnp.float32),
          pltpu.VMEM((GCH, 2 * DO), jnp.float32),
          pltpu.SemaphoreType.DMA,
          pltpu.SemaphoreType.DMA,
          pltpu.VMEM_SHARED((16 * CELLS,), jnp.int32),
          pltpu.VMEM_SHARED((16 * NB * 16,), jnp.int32),
      ],
  )
  def k(ei_hbm, m_hbm, ef_hbm, dstall, cells, cnts, stage, cbuf,
        deid, drow, acc, mbuf0, mbuf1, semg0, semg1, sp_cells, sp_cnts):
    c = lax.axis_index("c")
    s = lax.axis_index("s")
    wid = s * NC + c
    izero = jnp.zeros((16,), jnp.int32)
    negv = jnp.full((16,), NEG, jnp.float32)
    iota = lax.iota(jnp.int32, 16)
    DUMMY = 2 * RNG + 8

    def initc(i, _):
      cnts[pl.ds(i * 16, 16)] = izero
      return _

    lax.fori_loop(0, NB, initc, 0)

    def inita(i, _):
      for kk in range(DO // 16):
        acc[i, pl.ds(kk * 16, 16)] = negv
      return _

    lax.fori_loop(0, 2 * RNG + 16, inita, 0)

    base = wid * EPW
    pltpu.sync_copy(ei_hbm.at[1, pl.ds(base, EPW)], dstall)

    def scan(i, eids):
      v = dstall[pl.ds(i * 16, 16)]
      b = (v * 13108) >> 22
      doff = v - b * RNG
      penc = (doff << 19) + eids
      cidx = (b << 4) + iota
      cnt = plsc.load_gather(cnts, [cidx])
      pos = cidx * CAPL + jnp.minimum(cnt, CAPL - 1)
      plsc.store_scatter(cells, [pos], penc)
      plsc.store_scatter(cnts, [cidx], cnt + 1)
      return eids + 16

    lax.fori_loop(0, EPW // 16, scan, base + iota)

    pltpu.sync_copy(cells, sp_cells.at[pl.ds(s * CELLS, CELLS)])
    pltpu.sync_copy(cnts, sp_cnts.at[pl.ds(s * (NB * 16), NB * 16)])
    plsc.subcore_barrier()

    for bo in range(2):
      b = 2 * s + bo
      roff = bo * RNG

      def per_worker(s2, dtotal):
        pltpu.sync_copy(
            sp_cells.at[pl.ds(s2 * CELLS + b * (16 * CAPL), 16 * CAPL)],
            stage)
        pltpu.sync_copy(sp_cnts.at[pl.ds(s2 * (NB * 16) + b * 16, 16)], cbuf)
        cvec = cbuf[...]
        for l in range(16):
          cnt = jnp.minimum(cvec[l], CAPL)
          nv = (cnt + 15) >> 4

          def vec(kv, _, l=l, cnt=cnt, dtotal=dtotal):
            off = kv * 16
            w = stage[pl.ds(l * CAPL + off, 16)]
            msk = (off + iota) < cnt
            idx = dtotal + off + iota
            ev = w & MASK19
            p = ev >> 1
            row = jnp.where(p < E // 4, p << 1, ((p - E // 4) << 1) + 1)
            plsc.store_scatter(deid, [idx], row, mask=msk)
            plsc.store_scatter(drow, [idx],
                              (w >> 19) + roff + ((ev & 1) << 12), mask=msk)
            return _

          lax.fori_loop(0, nv, vec, 0)
          dtotal = jnp.minimum(dtotal + cnt, DCAP - CH)
        return dtotal

      dtotal = lax.fori_loop(0, 16, per_worker, jnp.int32(0))

      for kv in range(CH // 16):
        idx = dtotal + kv * 16 + iota
        plsc.store_scatter(deid, [idx], izero)
        plsc.store_scatter(drow, [idx], izero + DUMMY)

      ngr = (dtotal + GCH - 1) // GCH

      def fire(jc, mbuf, semg):
        pltpu.async_copy(m_hbm.at[deid.at[pl.ds(jc * GCH, GCH)]], mbuf, semg)

      def process(jc, mbuf, semg):
        pltpu.make_async_copy(m_hbm.at[deid.at[pl.ds(0, GCH)]], mbuf,
                              semg).wait()

        def grp(q, _):
          dvec = drow[pl.ds(jc * GCH + q * 16, 16)]
          for j in range(16):
            dvj = dvec[j]
            r = dvj & 4095
            par = dvj >> 12
            for kk in range(DO // 16):
              sl = pl.ds(kk * 16, 16)
              msl = pl.ds(par * DO + kk * 16, 16)
              acc[r, sl] = jnp.maximum(acc[r, sl], mbuf[q * 16 + j, msl])
          return _

        lax.fori_loop(0, GCH // 16, grp, 0)

      fire(0, mbuf0, semg0)

      def pstep(jc, carry):
        even = (jc & 1) == 0

        @pl.when(jc + 1 < ngr)
        def _prefetch():
          @pl.when(even)
          def _pf_even():
            fire(jc + 1, mbuf1, semg1)

          @pl.when(jnp.logical_not(even))
          def _pf_odd():
            fire(jc + 1, mbuf0, semg0)

        @pl.when(even)
        def _proc_even():
          process(jc, mbuf0, semg0)

        @pl.when(jnp.logical_not(even))
        def _proc_odd():
          process(jc, mbuf1, semg1)

        return carry

      lax.fori_loop(0, ngr, pstep, 0)
      pltpu.sync_copy(acc.at[pl.ds(roff, RNG)],
                      ef_hbm.at[c, pl.ds(b * RNG, RNG)])

  return k(dst, M)


def _tc_h(efp, Wc1, bc1, Wc2, bc2):
  RB = 1000
  nb = N // RB

  def body(e_ref, w1_ref, b1_ref, w2_ref, b2_ref, o_ref):
    e = jnp.maximum(e_ref[0], e_ref[1])
    ef = jnp.where(e == NEG, 0.0, e)
    cmid = jnp.maximum(
        jnp.dot(ef, w1_ref[...], preferred_element_type=jnp.float32)
        + b1_ref[...], 0.0)
    z = jnp.dot(cmid, w2_ref[...], preferred_element_type=jnp.float32) + b2_ref[...]
    o_ref[...] = jax.nn.sigmoid(z)

  return pl.pallas_call(
      body,
      grid=(nb,),
      in_specs=[
          pl.BlockSpec((NC, RB, DO), lambda i: (0, i, 0)),
          pl.BlockSpec((DO, DH), lambda i: (0, 0)),
          pl.BlockSpec((1, DH), lambda i: (0, 0)),
          pl.BlockSpec((DH, 1), lambda i: (0, 0)),
          pl.BlockSpec((1, 1), lambda i: (0, 0)),
      ],
      out_specs=pl.BlockSpec((RB, 1), lambda i: (i, 0)),
      out_shape=jax.ShapeDtypeStruct((N, 1), jnp.float32),
  )(efp, Wc1, bc1.reshape(1, DH), Wc2, bc2.reshape(1, 1))


def kernel(x, edge_index, W1, b1, We1, be1, We2, be2, Wc1, bc1, Wc2, bc2):
  degp = _sc_hist(edge_index)
  h, g, dinv = _tc_b(x, W1, degp)
  accp = _sc_scatter_add(edge_index, g)
  P, Q = _tc_d(accp, h, dinv, b1, We1, be1)
  U2 = _sc_edge_u(edge_index, P, Q)
  M4 = _tc_f(U2, We2, be2)
  efp = _sc_segmax(edge_index, M4.reshape(E // 2, 2 * DO))
  out = _tc_h(efp[:, :N], Wc1, bc1, Wc2, bc2)
  return jnp.squeeze(out, axis=-1)
